# baseline (speedup 1.0000x reference)
import functools

import jax
import jax.numpy as jnp
from jax import lax
from jax.experimental import pallas as pl
from jax.experimental.pallas import tpu as pltpu
from jax.experimental.pallas import tpu_sc as plsc

N = 10000
E = 320000
D = 128
G = 64
NCLASS = 10

RB = 2000
CP = 10240

_NSUB = 32
_CHUNK = 80
_NCHUNK = 128
_EDGES_PER_SUB = _CHUNK * _NCHUNK
_PAD = _EDGES_PER_SUB - E // _NSUB
_NACC = N + _PAD
_ROWS_PER_SUB = _NACC // 16
_ZROWS = 80
_OROWS = 80


def _sc_scatter(hw, src, dst):
    mesh = plsc.VectorSubcoreMesh(core_axis_name="c", subcore_axis_name="s")

    @functools.partial(
        pl.kernel,
        out_type=jax.ShapeDtypeStruct((2, N, D), jnp.float32),
        mesh=mesh,
        scratch_types=[
            pltpu.VMEM((_CHUNK,), jnp.int32),
            pltpu.VMEM((_CHUNK,), jnp.int32),
            pltpu.VMEM((_CHUNK,), jnp.int32),
            pltpu.VMEM((_CHUNK,), jnp.int32),
            pltpu.VMEM((_CHUNK, D), jnp.float32),
            pltpu.VMEM((_CHUNK, D), jnp.float32),
            pltpu.VMEM((_ZROWS, D), jnp.float32),
            pltpu.VMEM_SHARED((_NACC, D), jnp.float32),
            pltpu.SemaphoreType.DMA,
            pltpu.SemaphoreType.DMA,
            pltpu.SemaphoreType.DMA,
            pltpu.SemaphoreType.DMA,
        ],
    )
    def k(hw_hbm, src_hbm, dst_hbm, out_hbm, si0, di0, si1, di1, rows0,
          rows1, zbuf, acc, sg0, sg1, sx0, sx1):
        c = lax.axis_index("c")
        s = lax.axis_index("s")
        wid = c * 16 + s

        zv = jnp.zeros((16,), jnp.float32)

        def zrow(r, carry):
            for l in range(D // 16):
                zbuf[r, pl.ds(l * 16, 16)] = zv
            return carry

        lax.fori_loop(0, _ZROWS, zrow, 0)
        row0 = s * _ROWS_PER_SUB
        for j in range(_ROWS_PER_SUB // _ZROWS):
            pltpu.sync_copy(zbuf, acc.at[pl.ds(row0 + j * _ZROWS, _ZROWS)])
        plsc.subcore_barrier()

        ebase = wid * _EDGES_PER_SUB
        npairs = _NCHUNK // 2

        def sslice(ch):
            return src_hbm.at[pl.ds(ebase + ch * _CHUNK, _CHUNK)]

        def dslice(ch):
            return dst_hbm.at[pl.ds(ebase + ch * _CHUNK, _CHUNK)]

        pltpu.sync_copy(sslice(0), si0)
        pltpu.sync_copy(dslice(0), di0)
        pltpu.async_copy(hw_hbm.at[si0], rows0, sg0)
        pltpu.async_copy(sslice(1), si1, sx0)
        pltpu.async_copy(dslice(1), di1, sx0)

        def body(p, carry):
            ch0 = p * 2
            pltpu.make_async_copy(sslice(ch0 + 1), si1, sx0).wait()
            pltpu.make_async_copy(dslice(ch0 + 1), di1, sx0).wait()
            pltpu.async_copy(hw_hbm.at[si1], rows1, sg1)
            pltpu.make_async_copy(hw_hbm.at[si0], rows0, sg0).wait()
            pltpu.sync_copy(rows0, acc.at[di0], add=True)

            @pl.when(p < npairs - 1)
            def _():
                pltpu.async_copy(sslice(ch0 + 2), si0, sx1)
                pltpu.async_copy(dslice(ch0 + 2), di0, sx1)

            pltpu.make_async_copy(hw_hbm.at[si1], rows1, sg1).wait()
            pltpu.sync_copy(rows1, acc.at[di1], add=True)

            @pl.when(p < npairs - 1)
            def _():
                pltpu.make_async_copy(sslice(ch0 + 2), si0, sx1).wait()
                pltpu.make_async_copy(dslice(ch0 + 2), di0, sx1).wait()
                pltpu.async_copy(hw_hbm.at[si0], rows0, sg0)
                pltpu.async_copy(sslice(ch0 + 3), si1, sx0)
                pltpu.async_copy(dslice(ch0 + 3), di1, sx0)

            return carry

        lax.fori_loop(0, npairs, body, 0)

        if _NCHUNK % 2 == 1:
            last = _NCHUNK - 1
            pltpu.sync_copy(sslice(last), si0)
            pltpu.sync_copy(dslice(last), di0)
            pltpu.async_copy(hw_hbm.at[si0], rows0, sg0).wait()
            pltpu.sync_copy(rows0, acc.at[di0], add=True)

        plsc.subcore_barrier()

        @pl.when(s < 15)
        def _():
            for j in range(_ROWS_PER_SUB // _ZROWS):
                r = row0 + j * _ZROWS
                pltpu.sync_copy(acc.at[pl.ds(r, _ZROWS)],
                                out_hbm.at[c, pl.ds(r, _ZROWS)])

        @pl.when(s == 15)
        def _():
            for j in range((N - 15 * _ROWS_PER_SUB) // _OROWS):
                r = 15 * _ROWS_PER_SUB + j * _OROWS
                pltpu.sync_copy(acc.at[pl.ds(r, _OROWS)],
                                out_hbm.at[c, pl.ds(r, _OROWS)])

    return k(hw, src, dst)


def _mm_embed(x, W_emb, b_emb, Wg0):
    def body(x_ref, we_ref, be_ref, wg_ref, o_ref):
        h = jnp.dot(x_ref[...], we_ref[...], preferred_element_type=jnp.float32)
        h = h + be_ref[...]
        o_ref[...] = jnp.dot(h, wg_ref[...], preferred_element_type=jnp.float32)

    return pl.pallas_call(
        body,
        grid=(N // RB,),
        in_specs=[
            pl.BlockSpec((RB, D), lambda i: (i, 0)),
            pl.BlockSpec((D, D), lambda i: (0, 0)),
            pl.BlockSpec((1, D), lambda i: (0, 0)),
            pl.BlockSpec((D, D), lambda i: (0, 0)),
        ],
        out_specs=pl.BlockSpec((RB, D), lambda i: (i, 0)),
        out_shape=jax.ShapeDtypeStruct((N, D), jnp.float32),
    )(x, W_emb, b_emb.reshape(1, D), Wg0)


def _mm_mid(parts, bg, Wg):
    def body(p_ref, bg_ref, wg_ref, o_ref):
        h = jax.nn.relu(p_ref[0] + p_ref[1] + bg_ref[...])
        o_ref[...] = jnp.dot(h, wg_ref[...], preferred_element_type=jnp.float32)

    return pl.pallas_call(
        body,
        grid=(N // RB,),
        in_specs=[
            pl.BlockSpec((2, RB, D), lambda i: (0, i, 0)),
            pl.BlockSpec((1, D), lambda i: (0, 0)),
            pl.BlockSpec((D, D), lambda i: (0, 0)),
        ],
        out_specs=pl.BlockSpec((RB, D), lambda i: (i, 0)),
        out_shape=jax.ShapeDtypeStruct((N, D), jnp.float32),
    )(parts, bg.reshape(1, D), Wg)


def _finalize(parts, bg3, p_unit):
    def body(p_ref, bg_ref, pu_ref, h_ref, s_ref):
        h = jax.nn.relu(p_ref[0] + p_ref[1] + bg_ref[...])
        h_ref[...] = h
        s_ref[...] = jnp.tanh(
            jnp.dot(h, pu_ref[...], preferred_element_type=jnp.float32))

    return pl.pallas_call(
        body,
        grid=(N // RB,),
        in_specs=[
            pl.BlockSpec((2, RB, D), lambda i: (0, i, 0)),
            pl.BlockSpec((1, D), lambda i: (0, 0)),
            pl.BlockSpec((D, 1), lambda i: (0, 0)),
        ],
        out_specs=[
            pl.BlockSpec((RB, D), lambda i: (i, 0)),
            pl.BlockSpec((RB, 1), lambda i: (i, 0)),
        ],
        out_shape=[
            jax.ShapeDtypeStruct((N, D), jnp.float32),
            jax.ShapeDtypeStruct((N, 1), jnp.float32),
        ],
    )(parts, bg3.reshape(1, D), p_unit)


def _rank(rmin, rmax, cmin, cmax, s_col, b_col, i_col, s_row, b_row, i_row):
    nchunks = CP // 128

    def body(rmin_ref, rmax_ref, cmin_ref, cmax_ref,
             sc_ref, bc_ref, ic_ref, sr_ref, br_ref, ir_ref, w_ref):
        pid = pl.program_id(0)
        blo = rmin_ref[pid]
        bhi = rmax_ref[pid]
        s_c = sc_ref[...]
        b_c = bc_ref[...]
        i_c = ic_ref[...]

        def cbody(cc, carry):
            rank, cnt = carry
            active = jnp.logical_not(
                (cmin_ref[cc] > bhi) | (cmax_ref[cc] < blo))

            def compute(carry):
                rank, cnt = carry
                s_r = sr_ref[:, pl.ds(cc * 128, 128)]
                b_r = br_ref[:, pl.ds(cc * 128, 128)]
                i_r = ir_ref[:, pl.ds(cc * 128, 128)]
                same = b_c == b_r
                beats = (s_r > s_c) | ((s_r == s_c) & (i_r < i_c))
                rank = rank + jnp.sum((same & beats).astype(jnp.float32),
                                      axis=1, keepdims=True)
                cnt = cnt + jnp.sum(same.astype(jnp.float32),
                                    axis=1, keepdims=True)
                return rank, cnt

            return lax.cond(active, compute, lambda c: c, (rank, cnt))

        zero = jnp.zeros((RB, 1), jnp.float32)
        rank, cnt = lax.fori_loop(0, nchunks, cbody, (zero, zero))
        k = jnp.floor((7.0 * cnt + 9.0) / 10.0)
        keep = rank < k
        w_ref[...] = jnp.where(keep, s_c / jnp.maximum(k, 1.0), 0.0)

    return pl.pallas_call(
        body,
        grid=(N // RB,),
        in_specs=[
            pl.BlockSpec(memory_space=pltpu.SMEM),
            pl.BlockSpec(memory_space=pltpu.SMEM),
            pl.BlockSpec(memory_space=pltpu.SMEM),
            pl.BlockSpec(memory_space=pltpu.SMEM),
            pl.BlockSpec((RB, 1), lambda i: (i, 0)),
            pl.BlockSpec((RB, 1), lambda i: (i, 0)),
            pl.BlockSpec((RB, 1), lambda i: (i, 0)),
            pl.BlockSpec((1, CP), lambda i: (0, 0)),
            pl.BlockSpec((1, CP), lambda i: (0, 0)),
            pl.BlockSpec((1, CP), lambda i: (0, 0)),
        ],
        out_specs=pl.BlockSpec((RB, 1), lambda i: (i, 0)),
        out_shape=jax.ShapeDtypeStruct((N, 1), jnp.float32),
    )(rmin, rmax, cmin, cmax, s_col, b_col, i_col, s_row, b_row, i_row)


def _pool_mlp(h, w_col, b_col, Wm0, bm0, Wm1, bm1, Wm2, bm2):
    def body(h_ref, w_ref, b_ref, wm0_ref, bm0_ref, wm1_ref, bm1_ref,
             wm2_ref, bm2_ref, o_ref, acc):
        pid = pl.program_id(0)

        @pl.when(pid == 0)
        def _():
            acc[...] = jnp.zeros_like(acc)

        gids = lax.broadcasted_iota(jnp.int32, (1, G), 1).astype(jnp.float32)
        onehot = (b_ref[...] == gids).astype(jnp.float32)
        wh = h_ref[...] * w_ref[...]
        acc[...] += lax.dot_general(
            onehot, wh, (((0,), (0,)), ((), ())),
            preferred_element_type=jnp.float32)

        @pl.when(pid == N // RB - 1)
        def _():
            hg = acc[...]
            z = jax.nn.relu(
                jnp.dot(hg, wm0_ref[...], preferred_element_type=jnp.float32)
                + bm0_ref[...])
            z = jax.nn.relu(
                jnp.dot(z, wm1_ref[...], preferred_element_type=jnp.float32)
                + bm1_ref[...])
            o_ref[...] = (
                jnp.dot(z, wm2_ref[...], preferred_element_type=jnp.float32)
                + bm2_ref[...])

    return pl.pallas_call(
        body,
        grid=(N // RB,),
        in_specs=[
            pl.BlockSpec((RB, D), lambda i: (i, 0)),
            pl.BlockSpec((RB, 1), lambda i: (i, 0)),
            pl.BlockSpec((RB, 1), lambda i: (i, 0)),
            pl.BlockSpec((D, D // 2), lambda i: (0, 0)),
            pl.BlockSpec((1, D // 2), lambda i: (0, 0)),
            pl.BlockSpec((D // 2, D // 4), lambda i: (0, 0)),
            pl.BlockSpec((1, D // 4), lambda i: (0, 0)),
            pl.BlockSpec((D // 4, NCLASS), lambda i: (0, 0)),
            pl.BlockSpec((1, NCLASS), lambda i: (0, 0)),
        ],
        out_specs=pl.BlockSpec((G, NCLASS), lambda i: (0, 0)),
        out_shape=jax.ShapeDtypeStruct((G, NCLASS), jnp.float32),
        scratch_shapes=[pltpu.VMEM((G, D), jnp.float32)],
    )(h, w_col, b_col, Wm0, bm0.reshape(1, -1), Wm1, bm1.reshape(1, -1),
      Wm2, bm2.reshape(1, -1))


def kernel(x, edge_index, batch, W_emb, b_emb, Wg0, bg0, Wg1, bg1, Wg2, bg2,
           Wg3, bg3, p_topk, Wm0, bm0, Wm1, bm1, Wm2, bm2):
    src = jnp.pad(edge_index[0].reshape(_NSUB, E // _NSUB),
                  ((0, 0), (0, _PAD))).reshape(-1)
    padrows = jnp.broadcast_to(N + jnp.arange(_PAD, dtype=jnp.int32),
                               (_NSUB, _PAD))
    dst = jnp.concatenate(
        [edge_index[1].reshape(_NSUB, E // _NSUB), padrows], axis=1).reshape(-1)

    hw = _mm_embed(x, W_emb, b_emb, Wg0)
    for Wg, bg in ((Wg1, bg1), (Wg2, bg2), (Wg3, bg3)):
        parts = _sc_scatter(hw, src, dst)
        hw = _mm_mid(parts, bg, Wg)
    parts = _sc_scatter(hw, src, dst)

    p_unit = (p_topk / jnp.linalg.norm(p_topk)).reshape(D, 1)
    h3, s_col = _finalize(parts, bg3, p_unit)

    bf = batch.astype(jnp.float32)
    b_col = bf.reshape(N, 1)
    i_col = jnp.arange(N, dtype=jnp.float32).reshape(N, 1)
    s_row = jnp.concatenate(
        [s_col.reshape(1, N), jnp.full((1, CP - N), -2.0, jnp.float32)], axis=1)
    b_row = jnp.concatenate(
        [bf.reshape(1, N), jnp.full((1, CP - N), -1.0, jnp.float32)], axis=1)
    i_row = jnp.arange(CP, dtype=jnp.float32).reshape(1, CP)
    bi = batch.astype(jnp.int32)
    rmin = bi.reshape(N // RB, RB).min(axis=1)
    rmax = bi.reshape(N // RB, RB).max(axis=1)
    bp = jnp.concatenate([bi, jnp.full((CP - N,), -1, jnp.int32)])
    cmin = bp.reshape(CP // 128, 128).min(axis=1)
    cmax = bp.reshape(CP // 128, 128).max(axis=1)

    w_col = _rank(rmin, rmax, cmin, cmax, s_col, b_col, i_col,
                  s_row, b_row, i_row)
    return _pool_mlp(h3, w_col, b_col, Wm0, bm0, Wm1, bm1, Wm2, bm2)


# speedup vs baseline: 1.0000x; 1.0000x over previous
import functools

import jax
import jax.numpy as jnp
from jax import lax
from jax.experimental import pallas as pl
from jax.experimental.pallas import tpu as pltpu
from jax.experimental.pallas import tpu_sc as plsc

N = 10000
E = 320000
D = 128
G = 64
NCLASS = 10

RB = 2000
CP = 10240

_NSUB = 32
_CHUNK = 80
_NCHUNK = 128
_EDGES_PER_SUB = _CHUNK * _NCHUNK
_PAD = _EDGES_PER_SUB - E // _NSUB
_NACC = N + _PAD
_ROWS_PER_SUB = _NACC // 16
_ZROWS = 80
_OROWS = 80


def _sc_scatter(hw, src, dst):
    mesh = plsc.VectorSubcoreMesh(core_axis_name="c", subcore_axis_name="s")

    @functools.partial(
        pl.kernel,
        out_type=jax.ShapeDtypeStruct((2, N, D), jnp.float32),
        mesh=mesh,
        scratch_types=[
            pltpu.VMEM((_CHUNK,), jnp.int32),
            pltpu.VMEM((_CHUNK,), jnp.int32),
            pltpu.VMEM((_CHUNK,), jnp.int32),
            pltpu.VMEM((_CHUNK,), jnp.int32),
            pltpu.VMEM((_CHUNK, D), jnp.float32),
            pltpu.VMEM((_CHUNK, D), jnp.float32),
            pltpu.VMEM((_ZROWS, D), jnp.float32),
            pltpu.VMEM_SHARED((_NACC, D), jnp.float32),
            pltpu.SemaphoreType.DMA,
            pltpu.SemaphoreType.DMA,
            pltpu.SemaphoreType.DMA,
            pltpu.SemaphoreType.DMA,
        ],
    )
    def k(hw_hbm, src_hbm, dst_hbm, out_hbm, si0, di0, si1, di1, rows0,
          rows1, zbuf, acc, sg0, sg1, sx0, sx1):
        c = lax.axis_index("c")
        s = lax.axis_index("s")
        wid = c * 16 + s

        zv = jnp.zeros((16,), jnp.float32)

        def zrow(r, carry):
            for l in range(D // 16):
                zbuf[r, pl.ds(l * 16, 16)] = zv
            return carry

        lax.fori_loop(0, _ZROWS, zrow, 0)
        row0 = s * _ROWS_PER_SUB
        for j in range(_ROWS_PER_SUB // _ZROWS):
            pltpu.sync_copy(zbuf, acc.at[pl.ds(row0 + j * _ZROWS, _ZROWS)])
        plsc.subcore_barrier()

        ebase = wid * _EDGES_PER_SUB
        npairs = _NCHUNK // 2

        def sslice(ch):
            return src_hbm.at[pl.ds(ebase + ch * _CHUNK, _CHUNK)]

        def dslice(ch):
            return dst_hbm.at[pl.ds(ebase + ch * _CHUNK, _CHUNK)]

        pltpu.sync_copy(sslice(0), si0)
        pltpu.sync_copy(dslice(0), di0)
        pltpu.async_copy(hw_hbm.at[si0], rows0, sg0)
        pltpu.async_copy(sslice(1), si1, sx0)
        pltpu.async_copy(dslice(1), di1, sx0)

        def body(p, carry):
            ch0 = p * 2
            pltpu.make_async_copy(sslice(ch0 + 1), si1, sx0).wait()
            pltpu.make_async_copy(dslice(ch0 + 1), di1, sx0).wait()
            pltpu.async_copy(hw_hbm.at[si1], rows1, sg1)
            pltpu.make_async_copy(hw_hbm.at[si0], rows0, sg0).wait()
            pltpu.sync_copy(rows0, acc.at[di0], add=True)

            @pl.when(p < npairs - 1)
            def _():
                pltpu.async_copy(sslice(ch0 + 2), si0, sx1)
                pltpu.async_copy(dslice(ch0 + 2), di0, sx1)

            pltpu.make_async_copy(hw_hbm.at[si1], rows1, sg1).wait()
            pltpu.sync_copy(rows1, acc.at[di1], add=True)

            @pl.when(p < npairs - 1)
            def _():
                pltpu.make_async_copy(sslice(ch0 + 2), si0, sx1).wait()
                pltpu.make_async_copy(dslice(ch0 + 2), di0, sx1).wait()
                pltpu.async_copy(hw_hbm.at[si0], rows0, sg0)
                pltpu.async_copy(sslice(ch0 + 3), si1, sx0)
                pltpu.async_copy(dslice(ch0 + 3), di1, sx0)

            return carry

        lax.fori_loop(0, npairs, body, 0)

        if _NCHUNK % 2 == 1:
            last = _NCHUNK - 1
            pltpu.sync_copy(sslice(last), si0)
            pltpu.sync_copy(dslice(last), di0)
            pltpu.async_copy(hw_hbm.at[si0], rows0, sg0).wait()
            pltpu.sync_copy(rows0, acc.at[di0], add=True)

        plsc.subcore_barrier()

        @pl.when(s < 15)
        def _():
            for j in range(_ROWS_PER_SUB // _ZROWS):
                r = row0 + j * _ZROWS
                pltpu.sync_copy(acc.at[pl.ds(r, _ZROWS)],
                                out_hbm.at[c, pl.ds(r, _ZROWS)])

        @pl.when(s == 15)
        def _():
            for j in range((N - 15 * _ROWS_PER_SUB) // _OROWS):
                r = 15 * _ROWS_PER_SUB + j * _OROWS
                pltpu.sync_copy(acc.at[pl.ds(r, _OROWS)],
                                out_hbm.at[c, pl.ds(r, _OROWS)])

    return k(hw, src, dst)


def _mm_embed(x, W_emb, b_emb, Wg0):
    def body(x_ref, we_ref, be_ref, wg_ref, o_ref):
        h = jnp.dot(x_ref[...], we_ref[...], preferred_element_type=jnp.float32)
        h = h + be_ref[...]
        o_ref[...] = jnp.dot(h, wg_ref[...], preferred_element_type=jnp.float32)

    return pl.pallas_call(
        body,
        grid=(N // RB,),
        in_specs=[
            pl.BlockSpec((RB, D), lambda i: (i, 0)),
            pl.BlockSpec((D, D), lambda i: (0, 0)),
            pl.BlockSpec((1, D), lambda i: (0, 0)),
            pl.BlockSpec((D, D), lambda i: (0, 0)),
        ],
        out_specs=pl.BlockSpec((RB, D), lambda i: (i, 0)),
        out_shape=jax.ShapeDtypeStruct((N, D), jnp.float32),
    )(x, W_emb, b_emb.reshape(1, D), Wg0)


def _mm_mid(parts, bg, Wg):
    def body(p_ref, bg_ref, wg_ref, o_ref):
        h = jax.nn.relu(p_ref[0] + p_ref[1] + bg_ref[...])
        o_ref[...] = jnp.dot(h, wg_ref[...], preferred_element_type=jnp.float32)

    return pl.pallas_call(
        body,
        grid=(N // RB,),
        in_specs=[
            pl.BlockSpec((2, RB, D), lambda i: (0, i, 0)),
            pl.BlockSpec((1, D), lambda i: (0, 0)),
            pl.BlockSpec((D, D), lambda i: (0, 0)),
        ],
        out_specs=pl.BlockSpec((RB, D), lambda i: (i, 0)),
        out_shape=jax.ShapeDtypeStruct((N, D), jnp.float32),
    )(parts, bg.reshape(1, D), Wg)


def _finalize(parts, bg3, p_unit):
    def body(p_ref, bg_ref, pu_ref, h_ref, s_ref):
        h = jax.nn.relu(p_ref[0] + p_ref[1] + bg_ref[...])
        h_ref[...] = h
        s_ref[...] = jnp.tanh(
            jnp.dot(h, pu_ref[...], preferred_element_type=jnp.float32))

    return pl.pallas_call(
        body,
        grid=(N // RB,),
        in_specs=[
            pl.BlockSpec((2, RB, D), lambda i: (0, i, 0)),
            pl.BlockSpec((1, D), lambda i: (0, 0)),
            pl.BlockSpec((D, 1), lambda i: (0, 0)),
        ],
        out_specs=[
            pl.BlockSpec((RB, D), lambda i: (i, 0)),
            pl.BlockSpec((RB, 1), lambda i: (i, 0)),
        ],
        out_shape=[
            jax.ShapeDtypeStruct((N, D), jnp.float32),
            jax.ShapeDtypeStruct((N, 1), jnp.float32),
        ],
    )(parts, bg3.reshape(1, D), p_unit)


def _rank(rmin, rmax, cmin, cmax, s_col, b_col, i_col, s_row, b_row, i_row):
    nchunks = CP // 128

    def body(rmin_ref, rmax_ref, cmin_ref, cmax_ref,
             sc_ref, bc_ref, ic_ref, sr_ref, br_ref, ir_ref, w_ref):
        pid = pl.program_id(0)
        blo = rmin_ref[pid]
        bhi = rmax_ref[pid]
        s_c = sc_ref[...]
        b_c = bc_ref[...]
        i_c = ic_ref[...]

        def cbody(cc, carry):
            rank, cnt = carry
            active = jnp.logical_not(
                (cmin_ref[cc] > bhi) | (cmax_ref[cc] < blo))

            def compute(carry):
                rank, cnt = carry
                s_r = sr_ref[:, pl.ds(cc * 128, 128)]
                b_r = br_ref[:, pl.ds(cc * 128, 128)]
                i_r = ir_ref[:, pl.ds(cc * 128, 128)]
                same = b_c == b_r
                beats = (s_r > s_c) | ((s_r == s_c) & (i_r < i_c))
                rank = rank + jnp.sum((same & beats).astype(jnp.float32),
                                      axis=1, keepdims=True)
                cnt = cnt + jnp.sum(same.astype(jnp.float32),
                                    axis=1, keepdims=True)
                return rank, cnt

            return lax.cond(active, compute, lambda c: c, (rank, cnt))

        zero = jnp.zeros((RB, 1), jnp.float32)
        rank, cnt = lax.fori_loop(0, nchunks, cbody, (zero, zero))
        k = jnp.floor((7.0 * cnt + 9.0) / 10.0)
        keep = rank < k
        w_ref[...] = jnp.where(keep, s_c / jnp.maximum(k, 1.0), 0.0)

    return pl.pallas_call(
        body,
        grid=(N // RB,),
        in_specs=[
            pl.BlockSpec(memory_space=pltpu.SMEM),
            pl.BlockSpec(memory_space=pltpu.SMEM),
            pl.BlockSpec(memory_space=pltpu.SMEM),
            pl.BlockSpec(memory_space=pltpu.SMEM),
            pl.BlockSpec((RB, 1), lambda i: (i, 0)),
            pl.BlockSpec((RB, 1), lambda i: (i, 0)),
            pl.BlockSpec((RB, 1), lambda i: (i, 0)),
            pl.BlockSpec((1, CP), lambda i: (0, 0)),
            pl.BlockSpec((1, CP), lambda i: (0, 0)),
            pl.BlockSpec((1, CP), lambda i: (0, 0)),
        ],
        out_specs=pl.BlockSpec((RB, 1), lambda i: (i, 0)),
        out_shape=jax.ShapeDtypeStruct((N, 1), jnp.float32),
    )(rmin, rmax, cmin, cmax, s_col, b_col, i_col, s_row, b_row, i_row)


def _pool_mlp(h, w_col, b_col, Wm0, bm0, Wm1, bm1, Wm2, bm2):
    def body(h_ref, w_ref, b_ref, wm0_ref, bm0_ref, wm1_ref, bm1_ref,
             wm2_ref, bm2_ref, o_ref, acc):
        pid = pl.program_id(0)

        @pl.when(pid == 0)
        def _():
            acc[...] = jnp.zeros_like(acc)

        gids = lax.broadcasted_iota(jnp.int32, (1, G), 1).astype(jnp.float32)
        onehot = (b_ref[...] == gids).astype(jnp.float32)
        wh = h_ref[...] * w_ref[...]
        acc[...] += lax.dot_general(
            onehot, wh, (((0,), (0,)), ((), ())),
            preferred_element_type=jnp.float32)

        @pl.when(pid == N // RB - 1)
        def _():
            hg = acc[...]
            z = jax.nn.relu(
                jnp.dot(hg, wm0_ref[...], preferred_element_type=jnp.float32)
                + bm0_ref[...])
            z = jax.nn.relu(
                jnp.dot(z, wm1_ref[...], preferred_element_type=jnp.float32)
                + bm1_ref[...])
            o_ref[...] = (
                jnp.dot(z, wm2_ref[...], preferred_element_type=jnp.float32)
                + bm2_ref[...])

    return pl.pallas_call(
        body,
        grid=(N // RB,),
        in_specs=[
            pl.BlockSpec((RB, D), lambda i: (i, 0)),
            pl.BlockSpec((RB, 1), lambda i: (i, 0)),
            pl.BlockSpec((RB, 1), lambda i: (i, 0)),
            pl.BlockSpec((D, D // 2), lambda i: (0, 0)),
            pl.BlockSpec((1, D // 2), lambda i: (0, 0)),
            pl.BlockSpec((D // 2, D // 4), lambda i: (0, 0)),
            pl.BlockSpec((1, D // 4), lambda i: (0, 0)),
            pl.BlockSpec((D // 4, NCLASS), lambda i: (0, 0)),
            pl.BlockSpec((1, NCLASS), lambda i: (0, 0)),
        ],
        out_specs=pl.BlockSpec((G, NCLASS), lambda i: (0, 0)),
        out_shape=jax.ShapeDtypeStruct((G, NCLASS), jnp.float32),
        scratch_shapes=[pltpu.VMEM((G, D), jnp.float32)],
    )(h, w_col, b_col, Wm0, bm0.reshape(1, -1), Wm1, bm1.reshape(1, -1),
      Wm2, bm2.reshape(1, -1))


def kernel(x, edge_index, batch, W_emb, b_emb, Wg0, bg0, Wg1, bg1, Wg2, bg2,
           Wg3, bg3, p_topk, Wm0, bm0, Wm1, bm1, Wm2, bm2):
    src = jnp.pad(edge_index[0].reshape(_NSUB, E // _NSUB),
                  ((0, 0), (0, _PAD))).reshape(-1)
    padrows = (N + (jnp.arange(_NSUB, dtype=jnp.int32)[:, None] % 16) * 15
               + jnp.arange(_PAD, dtype=jnp.int32)[None, :] % 15)
    dst = jnp.concatenate(
        [edge_index[1].reshape(_NSUB, E // _NSUB), padrows], axis=1).reshape(-1)

    hw = _mm_embed(x, W_emb, b_emb, Wg0)
    for Wg, bg in ((Wg1, bg1), (Wg2, bg2), (Wg3, bg3)):
        parts = _sc_scatter(hw, src, dst)
        hw = _mm_mid(parts, bg, Wg)
    parts = _sc_scatter(hw, src, dst)

    p_unit = (p_topk / jnp.linalg.norm(p_topk)).reshape(D, 1)
    h3, s_col = _finalize(parts, bg3, p_unit)

    bf = batch.astype(jnp.float32)
    b_col = bf.reshape(N, 1)
    i_col = jnp.arange(N, dtype=jnp.float32).reshape(N, 1)
    s_row = jnp.concatenate(
        [s_col.reshape(1, N), jnp.full((1, CP - N), -2.0, jnp.float32)], axis=1)
    b_row = jnp.concatenate(
        [bf.reshape(1, N), jnp.full((1, CP - N), -1.0, jnp.float32)], axis=1)
    i_row = jnp.arange(CP, dtype=jnp.float32).reshape(1, CP)
    bi = batch.astype(jnp.int32)
    rmin = bi.reshape(N // RB, RB).min(axis=1)
    rmax = bi.reshape(N // RB, RB).max(axis=1)
    bp = jnp.concatenate([bi, jnp.full((CP - N,), -1, jnp.int32)])
    cmin = bp.reshape(CP // 128, 128).min(axis=1)
    cmax = bp.reshape(CP // 128, 128).max(axis=1)

    w_col = _rank(rmin, rmax, cmin, cmax, s_col, b_col, i_col,
                  s_row, b_row, i_row)
    return _pool_mlp(h3, w_col, b_col, Wm0, bm0, Wm1, bm1, Wm2, bm2)


# speedup vs baseline: 2.1089x; 2.1089x over previous
import functools

import jax
import jax.numpy as jnp
from jax import lax
from jax.experimental import pallas as pl
from jax.experimental.pallas import tpu as pltpu
from jax.experimental.pallas import tpu_sc as plsc

N = 10000
E = 320000
D = 128
G = 64
NCLASS = 10

RB = 2000
CP = 10240

_NSUB = 32
_EDGES_PER_SUB = E // _NSUB
_CHUNK = 80
_NCHUNK = _EDGES_PER_SUB // _CHUNK
_ROWS_PER_SUB = 624
_ZROWS = 208
_TAIL0 = 16 * _ROWS_PER_SUB
_TAILROWS = N - _TAIL0


def _sc_scatter(hw, src, dst):
    mesh = plsc.VectorSubcoreMesh(core_axis_name="c", subcore_axis_name="s")

    @functools.partial(
        pl.kernel,
        out_type=jax.ShapeDtypeStruct((2, N, D), jnp.float32),
        mesh=mesh,
        scratch_types=[
            pltpu.VMEM((_CHUNK,), jnp.int32),
            pltpu.VMEM((_CHUNK,), jnp.int32),
            pltpu.VMEM((_CHUNK,), jnp.int32),
            pltpu.VMEM((_CHUNK,), jnp.int32),
            pltpu.VMEM((_CHUNK, D), jnp.float32),
            pltpu.VMEM((_CHUNK, D), jnp.float32),
            pltpu.VMEM((_ZROWS, D), jnp.float32),
            pltpu.VMEM_SHARED((N, D), jnp.float32),
            pltpu.SemaphoreType.DMA,
            pltpu.SemaphoreType.DMA,
            pltpu.SemaphoreType.DMA,
            pltpu.SemaphoreType.DMA,
        ],
    )
    def k(hw_hbm, src_hbm, dst_hbm, out_hbm, si0, di0, si1, di1, rows0,
          rows1, zbuf, acc, sg0, sg1, sx0, sx1):
        c = lax.axis_index("c")
        s = lax.axis_index("s")
        wid = c * 16 + s

        zv = jnp.zeros((16,), jnp.float32)

        def zrow(r, carry):
            for l in range(D // 16):
                zbuf[r, pl.ds(l * 16, 16)] = zv
            return carry

        lax.fori_loop(0, _ZROWS, zrow, 0)
        row0 = s * _ROWS_PER_SUB
        for j in range(_ROWS_PER_SUB // _ZROWS):
            pltpu.sync_copy(zbuf, acc.at[pl.ds(row0 + j * _ZROWS, _ZROWS)])

        @pl.when(s == 15)
        def _():
            pltpu.sync_copy(zbuf.at[pl.ds(0, _TAILROWS)],
                            acc.at[pl.ds(_TAIL0, _TAILROWS)])

        plsc.subcore_barrier()

        ebase = wid * _EDGES_PER_SUB
        npairs = _NCHUNK // 2

        def sslice(ch):
            return src_hbm.at[pl.ds(ebase + ch * _CHUNK, _CHUNK)]

        def dslice(ch):
            return dst_hbm.at[pl.ds(ebase + ch * _CHUNK, _CHUNK)]

        pltpu.sync_copy(sslice(0), si0)
        pltpu.sync_copy(dslice(0), di0)
        pltpu.async_copy(hw_hbm.at[si0], rows0, sg0)
        pltpu.async_copy(sslice(1), si1, sx0)
        pltpu.async_copy(dslice(1), di1, sx0)

        def body(p, carry):
            ch0 = p * 2
            pltpu.make_async_copy(sslice(ch0 + 1), si1, sx0).wait()
            pltpu.make_async_copy(dslice(ch0 + 1), di1, sx0).wait()
            pltpu.async_copy(hw_hbm.at[si1], rows1, sg1)
            pltpu.make_async_copy(hw_hbm.at[si0], rows0, sg0).wait()
            pltpu.sync_copy(rows0, acc.at[di0], add=True)

            @pl.when(p < npairs - 1)
            def _():
                pltpu.async_copy(sslice(ch0 + 2), si0, sx1)
                pltpu.async_copy(dslice(ch0 + 2), di0, sx1)

            pltpu.make_async_copy(hw_hbm.at[si1], rows1, sg1).wait()
            pltpu.sync_copy(rows1, acc.at[di1], add=True)

            @pl.when(p < npairs - 1)
            def _():
                pltpu.make_async_copy(sslice(ch0 + 2), si0, sx1).wait()
                pltpu.make_async_copy(dslice(ch0 + 2), di0, sx1).wait()
                pltpu.async_copy(hw_hbm.at[si0], rows0, sg0)
                pltpu.async_copy(sslice(ch0 + 3), si1, sx0)
                pltpu.async_copy(dslice(ch0 + 3), di1, sx0)

            return carry

        lax.fori_loop(0, npairs, body, 0)

        if _NCHUNK % 2 == 1:
            last = _NCHUNK - 1
            pltpu.sync_copy(sslice(last), si0)
            pltpu.sync_copy(dslice(last), di0)
            pltpu.async_copy(hw_hbm.at[si0], rows0, sg0).wait()
            pltpu.sync_copy(rows0, acc.at[di0], add=True)

        plsc.subcore_barrier()

        for j in range(_ROWS_PER_SUB // _ZROWS):
            r = row0 + j * _ZROWS
            pltpu.sync_copy(acc.at[pl.ds(r, _ZROWS)],
                            out_hbm.at[c, pl.ds(r, _ZROWS)])

        @pl.when(s == 15)
        def _():
            pltpu.sync_copy(acc.at[pl.ds(_TAIL0, _TAILROWS)],
                            out_hbm.at[c, pl.ds(_TAIL0, _TAILROWS)])

    return k(hw, src, dst)


def _mm_embed(x, W_emb, b_emb, Wg0):
    def body(x_ref, we_ref, be_ref, wg_ref, o_ref):
        h = jnp.dot(x_ref[...], we_ref[...], preferred_element_type=jnp.float32)
        h = h + be_ref[...]
        o_ref[...] = jnp.dot(h, wg_ref[...], preferred_element_type=jnp.float32)

    return pl.pallas_call(
        body,
        grid=(N // RB,),
        in_specs=[
            pl.BlockSpec((RB, D), lambda i: (i, 0)),
            pl.BlockSpec((D, D), lambda i: (0, 0)),
            pl.BlockSpec((1, D), lambda i: (0, 0)),
            pl.BlockSpec((D, D), lambda i: (0, 0)),
        ],
        out_specs=pl.BlockSpec((RB, D), lambda i: (i, 0)),
        out_shape=jax.ShapeDtypeStruct((N, D), jnp.float32),
    )(x, W_emb, b_emb.reshape(1, D), Wg0)


def _mm_mid(parts, bg, Wg):
    def body(p_ref, bg_ref, wg_ref, o_ref):
        h = jax.nn.relu(p_ref[0] + p_ref[1] + bg_ref[...])
        o_ref[...] = jnp.dot(h, wg_ref[...], preferred_element_type=jnp.float32)

    return pl.pallas_call(
        body,
        grid=(N // RB,),
        in_specs=[
            pl.BlockSpec((2, RB, D), lambda i: (0, i, 0)),
            pl.BlockSpec((1, D), lambda i: (0, 0)),
            pl.BlockSpec((D, D), lambda i: (0, 0)),
        ],
        out_specs=pl.BlockSpec((RB, D), lambda i: (i, 0)),
        out_shape=jax.ShapeDtypeStruct((N, D), jnp.float32),
    )(parts, bg.reshape(1, D), Wg)


def _finalize(parts, bg3, p_unit):
    def body(p_ref, bg_ref, pu_ref, h_ref, s_ref):
        h = jax.nn.relu(p_ref[0] + p_ref[1] + bg_ref[...])
        h_ref[...] = h
        s_ref[...] = jnp.tanh(
            jnp.dot(h, pu_ref[...], preferred_element_type=jnp.float32))

    return pl.pallas_call(
        body,
        grid=(N // RB,),
        in_specs=[
            pl.BlockSpec((2, RB, D), lambda i: (0, i, 0)),
            pl.BlockSpec((1, D), lambda i: (0, 0)),
            pl.BlockSpec((D, 1), lambda i: (0, 0)),
        ],
        out_specs=[
            pl.BlockSpec((RB, D), lambda i: (i, 0)),
            pl.BlockSpec((RB, 1), lambda i: (i, 0)),
        ],
        out_shape=[
            jax.ShapeDtypeStruct((N, D), jnp.float32),
            jax.ShapeDtypeStruct((N, 1), jnp.float32),
        ],
    )(parts, bg3.reshape(1, D), p_unit)


def _rank(rmin, rmax, cmin, cmax, s_col, b_col, i_col, s_row, b_row, i_row):
    nchunks = CP // 128

    def body(rmin_ref, rmax_ref, cmin_ref, cmax_ref,
             sc_ref, bc_ref, ic_ref, sr_ref, br_ref, ir_ref, w_ref):
        pid = pl.program_id(0)
        blo = rmin_ref[pid]
        bhi = rmax_ref[pid]
        s_c = sc_ref[...]
        b_c = bc_ref[...]
        i_c = ic_ref[...]

        def cbody(cc, carry):
            rank, cnt = carry
            active = jnp.logical_not(
                (cmin_ref[cc] > bhi) | (cmax_ref[cc] < blo))

            def compute(carry):
                rank, cnt = carry
                s_r = sr_ref[:, pl.ds(cc * 128, 128)]
                b_r = br_ref[:, pl.ds(cc * 128, 128)]
                i_r = ir_ref[:, pl.ds(cc * 128, 128)]
                same = b_c == b_r
                beats = (s_r > s_c) | ((s_r == s_c) & (i_r < i_c))
                rank = rank + jnp.sum((same & beats).astype(jnp.float32),
                                      axis=1, keepdims=True)
                cnt = cnt + jnp.sum(same.astype(jnp.float32),
                                    axis=1, keepdims=True)
                return rank, cnt

            return lax.cond(active, compute, lambda c: c, (rank, cnt))

        zero = jnp.zeros((RB, 1), jnp.float32)
        rank, cnt = lax.fori_loop(0, nchunks, cbody, (zero, zero))
        k = jnp.floor((7.0 * cnt + 9.0) / 10.0)
        keep = rank < k
        w_ref[...] = jnp.where(keep, s_c / jnp.maximum(k, 1.0), 0.0)

    return pl.pallas_call(
        body,
        grid=(N // RB,),
        in_specs=[
            pl.BlockSpec(memory_space=pltpu.SMEM),
            pl.BlockSpec(memory_space=pltpu.SMEM),
            pl.BlockSpec(memory_space=pltpu.SMEM),
            pl.BlockSpec(memory_space=pltpu.SMEM),
            pl.BlockSpec((RB, 1), lambda i: (i, 0)),
            pl.BlockSpec((RB, 1), lambda i: (i, 0)),
            pl.BlockSpec((RB, 1), lambda i: (i, 0)),
            pl.BlockSpec((1, CP), lambda i: (0, 0)),
            pl.BlockSpec((1, CP), lambda i: (0, 0)),
            pl.BlockSpec((1, CP), lambda i: (0, 0)),
        ],
        out_specs=pl.BlockSpec((RB, 1), lambda i: (i, 0)),
        out_shape=jax.ShapeDtypeStruct((N, 1), jnp.float32),
    )(rmin, rmax, cmin, cmax, s_col, b_col, i_col, s_row, b_row, i_row)


def _pool_mlp(h, w_col, b_col, Wm0, bm0, Wm1, bm1, Wm2, bm2):
    def body(h_ref, w_ref, b_ref, wm0_ref, bm0_ref, wm1_ref, bm1_ref,
             wm2_ref, bm2_ref, o_ref, acc):
        pid = pl.program_id(0)

        @pl.when(pid == 0)
        def _():
            acc[...] = jnp.zeros_like(acc)

        gids = lax.broadcasted_iota(jnp.int32, (1, G), 1).astype(jnp.float32)
        onehot = (b_ref[...] == gids).astype(jnp.float32)
        wh = h_ref[...] * w_ref[...]
        acc[...] += lax.dot_general(
            onehot, wh, (((0,), (0,)), ((), ())),
            preferred_element_type=jnp.float32)

        @pl.when(pid == N // RB - 1)
        def _():
            hg = acc[...]
            z = jax.nn.relu(
                jnp.dot(hg, wm0_ref[...], preferred_element_type=jnp.float32)
                + bm0_ref[...])
            z = jax.nn.relu(
                jnp.dot(z, wm1_ref[...], preferred_element_type=jnp.float32)
                + bm1_ref[...])
            o_ref[...] = (
                jnp.dot(z, wm2_ref[...], preferred_element_type=jnp.float32)
                + bm2_ref[...])

    return pl.pallas_call(
        body,
        grid=(N // RB,),
        in_specs=[
            pl.BlockSpec((RB, D), lambda i: (i, 0)),
            pl.BlockSpec((RB, 1), lambda i: (i, 0)),
            pl.BlockSpec((RB, 1), lambda i: (i, 0)),
            pl.BlockSpec((D, D // 2), lambda i: (0, 0)),
            pl.BlockSpec((1, D // 2), lambda i: (0, 0)),
            pl.BlockSpec((D // 2, D // 4), lambda i: (0, 0)),
            pl.BlockSpec((1, D // 4), lambda i: (0, 0)),
            pl.BlockSpec((D // 4, NCLASS), lambda i: (0, 0)),
            pl.BlockSpec((1, NCLASS), lambda i: (0, 0)),
        ],
        out_specs=pl.BlockSpec((G, NCLASS), lambda i: (0, 0)),
        out_shape=jax.ShapeDtypeStruct((G, NCLASS), jnp.float32),
        scratch_shapes=[pltpu.VMEM((G, D), jnp.float32)],
    )(h, w_col, b_col, Wm0, bm0.reshape(1, -1), Wm1, bm1.reshape(1, -1),
      Wm2, bm2.reshape(1, -1))


def kernel(x, edge_index, batch, W_emb, b_emb, Wg0, bg0, Wg1, bg1, Wg2, bg2,
           Wg3, bg3, p_topk, Wm0, bm0, Wm1, bm1, Wm2, bm2):
    src = edge_index[0]
    dst = edge_index[1]

    hw = _mm_embed(x, W_emb, b_emb, Wg0)
    for Wg, bg in ((Wg1, bg1), (Wg2, bg2), (Wg3, bg3)):
        parts = _sc_scatter(hw, src, dst)
        hw = _mm_mid(parts, bg, Wg)
    parts = _sc_scatter(hw, src, dst)

    p_unit = (p_topk / jnp.linalg.norm(p_topk)).reshape(D, 1)
    h3, s_col = _finalize(parts, bg3, p_unit)

    bf = batch.astype(jnp.float32)
    b_col = bf.reshape(N, 1)
    i_col = jnp.arange(N, dtype=jnp.float32).reshape(N, 1)
    s_row = jnp.concatenate(
        [s_col.reshape(1, N), jnp.full((1, CP - N), -2.0, jnp.float32)], axis=1)
    b_row = jnp.concatenate(
        [bf.reshape(1, N), jnp.full((1, CP - N), -1.0, jnp.float32)], axis=1)
    i_row = jnp.arange(CP, dtype=jnp.float32).reshape(1, CP)
    bi = batch.astype(jnp.int32)
    rmin = bi.reshape(N // RB, RB).min(axis=1)
    rmax = bi.reshape(N // RB, RB).max(axis=1)
    bp = jnp.concatenate([bi, jnp.full((CP - N,), -1, jnp.int32)])
    cmin = bp.reshape(CP // 128, 128).min(axis=1)
    cmax = bp.reshape(CP // 128, 128).max(axis=1)

    w_col = _rank(rmin, rmax, cmin, cmax, s_col, b_col, i_col,
                  s_row, b_row, i_row)
    return _pool_mlp(h3, w_col, b_col, Wm0, bm0, Wm1, bm1, Wm2, bm2)


# speedup vs baseline: 2.6390x; 1.2514x over previous
import functools

import jax
import jax.numpy as jnp
from jax import lax
from jax.experimental import pallas as pl
from jax.experimental.pallas import tpu as pltpu
from jax.experimental.pallas import tpu_sc as plsc

N = 10000
E = 320000
D = 128
G = 64
NCLASS = 10

RB = 2000
CP = 10240

_NSUB = 32
_EDGES_PER_SUB = E // _NSUB
_CHUNK = 80
_NCHUNK = _EDGES_PER_SUB // _CHUNK
_ROWS_PER_SUB = 624
_ZROWS = 48
_OROWS = 208
_TAIL0 = 16 * _ROWS_PER_SUB
_TAILROWS = N - _TAIL0


def _sc_scatter(hw, packed_ei):
    mesh = plsc.VectorSubcoreMesh(core_axis_name="c", subcore_axis_name="s")

    @functools.partial(
        pl.kernel,
        out_type=jax.ShapeDtypeStruct((2, N, D), jnp.float32),
        mesh=mesh,
        scratch_types=(
            [pltpu.VMEM((2, _CHUNK), jnp.int32) for _ in range(8)]
            + [pltpu.VMEM((_CHUNK, D), jnp.float32) for _ in range(4)]
            + [pltpu.VMEM((_ZROWS, D), jnp.float32),
               pltpu.VMEM_SHARED((N, D), jnp.float32)]
            + [pltpu.SemaphoreType.DMA for _ in range(12)]
        ),
    )
    def k(hw_hbm, ei_hbm, out_hbm,
          ia0, ia1, ia2, ia3, ib0, ib1, ib2, ib3,
          r0, r1, r2, r3, zbuf, acc,
          sg0, sg1, sg2, sg3, sa0, sa1, sa2, sa3, sb0, sb1, sb2, sb3):
        ia = [ia0, ia1, ia2, ia3]
        ib = [ib0, ib1, ib2, ib3]
        rows = [r0, r1, r2, r3]
        sg = [sg0, sg1, sg2, sg3]
        sa = [sa0, sa1, sa2, sa3]
        sb = [sb0, sb1, sb2, sb3]
        c = lax.axis_index("c")
        s = lax.axis_index("s")
        wid = c * 16 + s

        zv = jnp.zeros((16,), jnp.float32)

        def zrow(r, carry):
            for l in range(D // 16):
                zbuf[r, pl.ds(l * 16, 16)] = zv
            return carry

        lax.fori_loop(0, _ZROWS, zrow, 0)
        row0 = s * _ROWS_PER_SUB
        for j in range(_ROWS_PER_SUB // _ZROWS):
            pltpu.sync_copy(zbuf, acc.at[pl.ds(row0 + j * _ZROWS, _ZROWS)])

        @pl.when(s == 15)
        def _():
            pltpu.sync_copy(zbuf.at[pl.ds(0, _TAILROWS)],
                            acc.at[pl.ds(_TAIL0, _TAILROWS)])

        plsc.subcore_barrier()

        pbase = wid * _NCHUNK
        ngroups = _NCHUNK // 8

        def plane(ch):
            return ei_hbm.at[pbase + ch]

        for l in range(4):
            pltpu.sync_copy(plane(l), ia[l])
            pltpu.async_copy(hw_hbm.at[ia[l].at[0]], rows[l], sg[l])
        for l in range(4):
            pltpu.async_copy(plane(4 + l), ib[l], sb[l])

        def body(g, carry):
            base = g * 8
            for l in range(4):
                pltpu.make_async_copy(hw_hbm.at[ia[l].at[0]],
                                      rows[l], sg[l]).wait()
                pltpu.sync_copy(rows[l], acc.at[ia[l].at[1]], add=True)
                pltpu.make_async_copy(plane(base + 4 + l), ib[l], sb[l]).wait()
                pltpu.async_copy(hw_hbm.at[ib[l].at[0]], rows[l], sg[l])
                pltpu.async_copy(plane(base + 8 + l), ia[l], sa[l])
            for l in range(4):
                pltpu.make_async_copy(hw_hbm.at[ib[l].at[0]],
                                      rows[l], sg[l]).wait()
                pltpu.sync_copy(rows[l], acc.at[ib[l].at[1]], add=True)

                @pl.when(g < ngroups - 1)
                def _():
                    pltpu.make_async_copy(plane(base + 8 + l),
                                          ia[l], sa[l]).wait()
                    pltpu.async_copy(hw_hbm.at[ia[l].at[0]], rows[l], sg[l])
                    pltpu.async_copy(plane(base + 12 + l), ib[l], sb[l])

            return carry

        lax.fori_loop(0, ngroups, body, 0)

        t0 = ngroups * 8
        for l in range(4):
            pltpu.make_async_copy(plane(t0 + l), ia[l], sa[l]).wait()
            pltpu.async_copy(hw_hbm.at[ia[l].at[0]], rows[l], sg[l])
        pltpu.sync_copy(plane(t0 + 4), ib0)
        for l in range(4):
            pltpu.make_async_copy(hw_hbm.at[ia[l].at[0]],
                                  rows[l], sg[l]).wait()
            pltpu.sync_copy(rows[l], acc.at[ia[l].at[1]], add=True)
        pltpu.async_copy(hw_hbm.at[ib0.at[0]], rows[0], sg[0]).wait()
        pltpu.sync_copy(rows[0], acc.at[ib0.at[1]], add=True)

        plsc.subcore_barrier()

        for j in range(_ROWS_PER_SUB // _OROWS):
            r = row0 + j * _OROWS
            pltpu.sync_copy(acc.at[pl.ds(r, _OROWS)],
                            out_hbm.at[c, pl.ds(r, _OROWS)])

        @pl.when(s == 15)
        def _():
            pltpu.sync_copy(acc.at[pl.ds(_TAIL0, _TAILROWS)],
                            out_hbm.at[c, pl.ds(_TAIL0, _TAILROWS)])

    return k(hw, packed_ei)


def _mm_embed(x, W_emb, b_emb, Wg0):
    def body(x_ref, we_ref, be_ref, wg_ref, o_ref):
        h = jnp.dot(x_ref[...], we_ref[...], preferred_element_type=jnp.float32)
        h = h + be_ref[...]
        o_ref[...] = jnp.dot(h, wg_ref[...], preferred_element_type=jnp.float32)

    return pl.pallas_call(
        body,
        grid=(N // RB,),
        in_specs=[
            pl.BlockSpec((RB, D), lambda i: (i, 0)),
            pl.BlockSpec((D, D), lambda i: (0, 0)),
            pl.BlockSpec((1, D), lambda i: (0, 0)),
            pl.BlockSpec((D, D), lambda i: (0, 0)),
        ],
        out_specs=pl.BlockSpec((RB, D), lambda i: (i, 0)),
        out_shape=jax.ShapeDtypeStruct((N, D), jnp.float32),
    )(x, W_emb, b_emb.reshape(1, D), Wg0)


def _mm_mid(parts, bg, Wg):
    def body(p_ref, bg_ref, wg_ref, o_ref):
        h = jax.nn.relu(p_ref[0] + p_ref[1] + bg_ref[...])
        o_ref[...] = jnp.dot(h, wg_ref[...], preferred_element_type=jnp.float32)

    return pl.pallas_call(
        body,
        grid=(N // RB,),
        in_specs=[
            pl.BlockSpec((2, RB, D), lambda i: (0, i, 0)),
            pl.BlockSpec((1, D), lambda i: (0, 0)),
            pl.BlockSpec((D, D), lambda i: (0, 0)),
        ],
        out_specs=pl.BlockSpec((RB, D), lambda i: (i, 0)),
        out_shape=jax.ShapeDtypeStruct((N, D), jnp.float32),
    )(parts, bg.reshape(1, D), Wg)


def _finalize(parts, bg3, p_unit):
    def body(p_ref, bg_ref, pu_ref, h_ref, s_ref):
        h = jax.nn.relu(p_ref[0] + p_ref[1] + bg_ref[...])
        h_ref[...] = h
        s_ref[...] = jnp.tanh(
            jnp.dot(h, pu_ref[...], preferred_element_type=jnp.float32))

    return pl.pallas_call(
        body,
        grid=(N // RB,),
        in_specs=[
            pl.BlockSpec((2, RB, D), lambda i: (0, i, 0)),
            pl.BlockSpec((1, D), lambda i: (0, 0)),
            pl.BlockSpec((D, 1), lambda i: (0, 0)),
        ],
        out_specs=[
            pl.BlockSpec((RB, D), lambda i: (i, 0)),
            pl.BlockSpec((RB, 1), lambda i: (i, 0)),
        ],
        out_shape=[
            jax.ShapeDtypeStruct((N, D), jnp.float32),
            jax.ShapeDtypeStruct((N, 1), jnp.float32),
        ],
    )(parts, bg3.reshape(1, D), p_unit)


def _rank(rmin, rmax, cmin, cmax, s_col, b_col, i_col, s_row, b_row, i_row):
    nchunks = CP // 128

    def body(rmin_ref, rmax_ref, cmin_ref, cmax_ref,
             sc_ref, bc_ref, ic_ref, sr_ref, br_ref, ir_ref, w_ref):
        pid = pl.program_id(0)
        blo = rmin_ref[pid]
        bhi = rmax_ref[pid]
        s_c = sc_ref[...]
        b_c = bc_ref[...]
        i_c = ic_ref[...]

        def cbody(cc, carry):
            rank, cnt = carry
            active = jnp.logical_not(
                (cmin_ref[cc] > bhi) | (cmax_ref[cc] < blo))

            def compute(carry):
                rank, cnt = carry
                s_r = sr_ref[:, pl.ds(cc * 128, 128)]
                b_r = br_ref[:, pl.ds(cc * 128, 128)]
                i_r = ir_ref[:, pl.ds(cc * 128, 128)]
                same = b_c == b_r
                beats = (s_r > s_c) | ((s_r == s_c) & (i_r < i_c))
                rank = rank + jnp.sum((same & beats).astype(jnp.float32),
                                      axis=1, keepdims=True)
                cnt = cnt + jnp.sum(same.astype(jnp.float32),
                                    axis=1, keepdims=True)
                return rank, cnt

            return lax.cond(active, compute, lambda c: c, (rank, cnt))

        zero = jnp.zeros((RB, 1), jnp.float32)
        rank, cnt = lax.fori_loop(0, nchunks, cbody, (zero, zero))
        k = jnp.floor((7.0 * cnt + 9.0) / 10.0)
        keep = rank < k
        w_ref[...] = jnp.where(keep, s_c / jnp.maximum(k, 1.0), 0.0)

    return pl.pallas_call(
        body,
        grid=(N // RB,),
        in_specs=[
            pl.BlockSpec(memory_space=pltpu.SMEM),
            pl.BlockSpec(memory_space=pltpu.SMEM),
            pl.BlockSpec(memory_space=pltpu.SMEM),
            pl.BlockSpec(memory_space=pltpu.SMEM),
            pl.BlockSpec((RB, 1), lambda i: (i, 0)),
            pl.BlockSpec((RB, 1), lambda i: (i, 0)),
            pl.BlockSpec((RB, 1), lambda i: (i, 0)),
            pl.BlockSpec((1, CP), lambda i: (0, 0)),
            pl.BlockSpec((1, CP), lambda i: (0, 0)),
            pl.BlockSpec((1, CP), lambda i: (0, 0)),
        ],
        out_specs=pl.BlockSpec((RB, 1), lambda i: (i, 0)),
        out_shape=jax.ShapeDtypeStruct((N, 1), jnp.float32),
    )(rmin, rmax, cmin, cmax, s_col, b_col, i_col, s_row, b_row, i_row)


def _pool_mlp(h, w_col, b_col, Wm0, bm0, Wm1, bm1, Wm2, bm2):
    def body(h_ref, w_ref, b_ref, wm0_ref, bm0_ref, wm1_ref, bm1_ref,
             wm2_ref, bm2_ref, o_ref, acc):
        pid = pl.program_id(0)

        @pl.when(pid == 0)
        def _():
            acc[...] = jnp.zeros_like(acc)

        gids = lax.broadcasted_iota(jnp.int32, (1, G), 1).astype(jnp.float32)
        onehot = (b_ref[...] == gids).astype(jnp.float32)
        wh = h_ref[...] * w_ref[...]
        acc[...] += lax.dot_general(
            onehot, wh, (((0,), (0,)), ((), ())),
            preferred_element_type=jnp.float32)

        @pl.when(pid == N // RB - 1)
        def _():
            hg = acc[...]
            z = jax.nn.relu(
                jnp.dot(hg, wm0_ref[...], preferred_element_type=jnp.float32)
                + bm0_ref[...])
            z = jax.nn.relu(
                jnp.dot(z, wm1_ref[...], preferred_element_type=jnp.float32)
                + bm1_ref[...])
            o_ref[...] = (
                jnp.dot(z, wm2_ref[...], preferred_element_type=jnp.float32)
                + bm2_ref[...])

    return pl.pallas_call(
        body,
        grid=(N // RB,),
        in_specs=[
            pl.BlockSpec((RB, D), lambda i: (i, 0)),
            pl.BlockSpec((RB, 1), lambda i: (i, 0)),
            pl.BlockSpec((RB, 1), lambda i: (i, 0)),
            pl.BlockSpec((D, D // 2), lambda i: (0, 0)),
            pl.BlockSpec((1, D // 2), lambda i: (0, 0)),
            pl.BlockSpec((D // 2, D // 4), lambda i: (0, 0)),
            pl.BlockSpec((1, D // 4), lambda i: (0, 0)),
            pl.BlockSpec((D // 4, NCLASS), lambda i: (0, 0)),
            pl.BlockSpec((1, NCLASS), lambda i: (0, 0)),
        ],
        out_specs=pl.BlockSpec((G, NCLASS), lambda i: (0, 0)),
        out_shape=jax.ShapeDtypeStruct((G, NCLASS), jnp.float32),
        scratch_shapes=[pltpu.VMEM((G, D), jnp.float32)],
    )(h, w_col, b_col, Wm0, bm0.reshape(1, -1), Wm1, bm1.reshape(1, -1),
      Wm2, bm2.reshape(1, -1))


def kernel(x, edge_index, batch, W_emb, b_emb, Wg0, bg0, Wg1, bg1, Wg2, bg2,
           Wg3, bg3, p_topk, Wm0, bm0, Wm1, bm1, Wm2, bm2):
    packed_ei = jnp.stack([edge_index[0].reshape(-1, _CHUNK),
                           edge_index[1].reshape(-1, _CHUNK)], axis=1)

    hw = _mm_embed(x, W_emb, b_emb, Wg0)
    for Wg, bg in ((Wg1, bg1), (Wg2, bg2), (Wg3, bg3)):
        parts = _sc_scatter(hw, packed_ei)
        hw = _mm_mid(parts, bg, Wg)
    parts = _sc_scatter(hw, packed_ei)

    p_unit = (p_topk / jnp.linalg.norm(p_topk)).reshape(D, 1)
    h3, s_col = _finalize(parts, bg3, p_unit)

    bf = batch.astype(jnp.float32)
    b_col = bf.reshape(N, 1)
    i_col = jnp.arange(N, dtype=jnp.float32).reshape(N, 1)
    s_row = jnp.concatenate(
        [s_col.reshape(1, N), jnp.full((1, CP - N), -2.0, jnp.float32)], axis=1)
    b_row = jnp.concatenate(
        [bf.reshape(1, N), jnp.full((1, CP - N), -1.0, jnp.float32)], axis=1)
    i_row = jnp.arange(CP, dtype=jnp.float32).reshape(1, CP)
    bi = batch.astype(jnp.int32)
    rmin = bi.reshape(N // RB, RB).min(axis=1)
    rmax = bi.reshape(N // RB, RB).max(axis=1)
    bp = jnp.concatenate([bi, jnp.full((CP - N,), -1, jnp.int32)])
    cmin = bp.reshape(CP // 128, 128).min(axis=1)
    cmax = bp.reshape(CP // 128, 128).max(axis=1)

    w_col = _rank(rmin, rmax, cmin, cmax, s_col, b_col, i_col,
                  s_row, b_row, i_row)
    return _pool_mlp(h3, w_col, b_col, Wm0, bm0, Wm1, bm1, Wm2, bm2)


# speedup vs baseline: 2.6398x; 1.0003x over previous
import functools

import jax
import jax.numpy as jnp
from jax import lax
from jax.experimental import pallas as pl
from jax.experimental.pallas import tpu as pltpu
from jax.experimental.pallas import tpu_sc as plsc

N = 10000
E = 320000
D = 128
G = 64
NCLASS = 10

RB = 2000
CP = 10240

_NSUB = 32
_EDGES_PER_SUB = E // _NSUB
_CHUNK = 80
_NCHUNK = _EDGES_PER_SUB // _CHUNK
_ROWS_PER_SUB = 624
_ZROWS = 48
_OROWS = 208
_TAIL0 = 16 * _ROWS_PER_SUB
_TAILROWS = N - _TAIL0


def _sc_scatter(hw, src, dst):
    mesh = plsc.VectorSubcoreMesh(core_axis_name="c", subcore_axis_name="s")

    @functools.partial(
        pl.kernel,
        out_type=jax.ShapeDtypeStruct((2, N, D), jnp.float32),
        mesh=mesh,
        scratch_types=(
            [pltpu.VMEM((_CHUNK,), jnp.int32) for _ in range(16)]
            + [pltpu.VMEM((_CHUNK, D), jnp.float32) for _ in range(4)]
            + [pltpu.VMEM((_ZROWS, D), jnp.float32),
               pltpu.VMEM_SHARED((N, D), jnp.float32)]
            + [pltpu.SemaphoreType.DMA for _ in range(12)]
        ),
    )
    def k(hw_hbm, src_hbm, dst_hbm, out_hbm,
          sa0_, sa1_, sa2_, sa3_, da0_, da1_, da2_, da3_,
          sb0_, sb1_, sb2_, sb3_, db0_, db1_, db2_, db3_,
          r0, r1, r2, r3, zbuf, acc,
          sg0, sg1, sg2, sg3, sa0, sa1, sa2, sa3, sb0, sb1, sb2, sb3):
        sia = [sa0_, sa1_, sa2_, sa3_]
        dia = [da0_, da1_, da2_, da3_]
        sib = [sb0_, sb1_, sb2_, sb3_]
        dib = [db0_, db1_, db2_, db3_]
        rows = [r0, r1, r2, r3]
        sg = [sg0, sg1, sg2, sg3]
        sa = [sa0, sa1, sa2, sa3]
        sb = [sb0, sb1, sb2, sb3]
        c = lax.axis_index("c")
        s = lax.axis_index("s")
        wid = c * 16 + s

        zv = jnp.zeros((16,), jnp.float32)

        def zrow(r, carry):
            for l in range(D // 16):
                zbuf[r, pl.ds(l * 16, 16)] = zv
            return carry

        lax.fori_loop(0, _ZROWS, zrow, 0)
        row0 = s * _ROWS_PER_SUB
        for j in range(_ROWS_PER_SUB // _ZROWS):
            pltpu.sync_copy(zbuf, acc.at[pl.ds(row0 + j * _ZROWS, _ZROWS)])

        @pl.when(s == 15)
        def _():
            pltpu.sync_copy(zbuf.at[pl.ds(0, _TAILROWS)],
                            acc.at[pl.ds(_TAIL0, _TAILROWS)])

        plsc.subcore_barrier()

        ebase = wid * _EDGES_PER_SUB
        ngroups = _NCHUNK // 8

        def sslice(ch):
            return src_hbm.at[pl.ds(ebase + ch * _CHUNK, _CHUNK)]

        def dslice(ch):
            return dst_hbm.at[pl.ds(ebase + ch * _CHUNK, _CHUNK)]

        def load_idx(ch, sref, dref, sem):
            pltpu.async_copy(sslice(ch), sref, sem)
            pltpu.async_copy(dslice(ch), dref, sem)

        def wait_idx(ch, sref, dref, sem):
            pltpu.make_async_copy(sslice(ch), sref, sem).wait()
            pltpu.make_async_copy(dslice(ch), dref, sem).wait()

        for l in range(4):
            pltpu.sync_copy(sslice(l), sia[l])
            pltpu.sync_copy(dslice(l), dia[l])
            pltpu.async_copy(hw_hbm.at[sia[l]], rows[l], sg[l])
        for l in range(4):
            load_idx(4 + l, sib[l], dib[l], sb[l])

        def body(g, carry):
            base = g * 8
            for l in range(4):
                pltpu.make_async_copy(hw_hbm.at[sia[l]], rows[l], sg[l]).wait()
                pltpu.sync_copy(rows[l], acc.at[dia[l]], add=True)
                wait_idx(base + 4 + l, sib[l], dib[l], sb[l])
                pltpu.async_copy(hw_hbm.at[sib[l]], rows[l], sg[l])
                load_idx(base + 8 + l, sia[l], dia[l], sa[l])
            for l in range(4):
                pltpu.make_async_copy(hw_hbm.at[sib[l]], rows[l], sg[l]).wait()
                pltpu.sync_copy(rows[l], acc.at[dib[l]], add=True)

                @pl.when(g < ngroups - 1)
                def _():
                    wait_idx(base + 8 + l, sia[l], dia[l], sa[l])
                    pltpu.async_copy(hw_hbm.at[sia[l]], rows[l], sg[l])
                    load_idx(base + 12 + l, sib[l], dib[l], sb[l])

            return carry

        lax.fori_loop(0, ngroups, body, 0)

        t0 = ngroups * 8
        for l in range(4):
            wait_idx(t0 + l, sia[l], dia[l], sa[l])
            pltpu.async_copy(hw_hbm.at[sia[l]], rows[l], sg[l])
        pltpu.sync_copy(sslice(t0 + 4), sib[0])
        pltpu.sync_copy(dslice(t0 + 4), dib[0])
        for l in range(4):
            pltpu.make_async_copy(hw_hbm.at[sia[l]], rows[l], sg[l]).wait()
            pltpu.sync_copy(rows[l], acc.at[dia[l]], add=True)
        pltpu.async_copy(hw_hbm.at[sib[0]], rows[0], sg[0]).wait()
        pltpu.sync_copy(rows[0], acc.at[dib[0]], add=True)

        plsc.subcore_barrier()

        for j in range(_ROWS_PER_SUB // _OROWS):
            r = row0 + j * _OROWS
            pltpu.sync_copy(acc.at[pl.ds(r, _OROWS)],
                            out_hbm.at[c, pl.ds(r, _OROWS)])

        @pl.when(s == 15)
        def _():
            pltpu.sync_copy(acc.at[pl.ds(_TAIL0, _TAILROWS)],
                            out_hbm.at[c, pl.ds(_TAIL0, _TAILROWS)])

    return k(hw, src, dst)


def _mm_embed(x, W_emb, b_emb, Wg0):
    def body(x_ref, we_ref, be_ref, wg_ref, o_ref):
        h = jnp.dot(x_ref[...], we_ref[...], preferred_element_type=jnp.float32)
        h = h + be_ref[...]
        o_ref[...] = jnp.dot(h, wg_ref[...], preferred_element_type=jnp.float32)

    return pl.pallas_call(
        body,
        grid=(N // RB,),
        in_specs=[
            pl.BlockSpec((RB, D), lambda i: (i, 0)),
            pl.BlockSpec((D, D), lambda i: (0, 0)),
            pl.BlockSpec((1, D), lambda i: (0, 0)),
            pl.BlockSpec((D, D), lambda i: (0, 0)),
        ],
        out_specs=pl.BlockSpec((RB, D), lambda i: (i, 0)),
        out_shape=jax.ShapeDtypeStruct((N, D), jnp.float32),
    )(x, W_emb, b_emb.reshape(1, D), Wg0)


def _mm_mid(parts, bg, Wg):
    def body(p_ref, bg_ref, wg_ref, o_ref):
        h = jax.nn.relu(p_ref[0] + p_ref[1] + bg_ref[...])
        o_ref[...] = jnp.dot(h, wg_ref[...], preferred_element_type=jnp.float32)

    return pl.pallas_call(
        body,
        grid=(N // RB,),
        in_specs=[
            pl.BlockSpec((2, RB, D), lambda i: (0, i, 0)),
            pl.BlockSpec((1, D), lambda i: (0, 0)),
            pl.BlockSpec((D, D), lambda i: (0, 0)),
        ],
        out_specs=pl.BlockSpec((RB, D), lambda i: (i, 0)),
        out_shape=jax.ShapeDtypeStruct((N, D), jnp.float32),
    )(parts, bg.reshape(1, D), Wg)


def _finalize(parts, bg3, p_unit):
    def body(p_ref, bg_ref, pu_ref, h_ref, s_ref):
        h = jax.nn.relu(p_ref[0] + p_ref[1] + bg_ref[...])
        h_ref[...] = h
        s_ref[...] = jnp.tanh(
            jnp.dot(h, pu_ref[...], preferred_element_type=jnp.float32))

    return pl.pallas_call(
        body,
        grid=(N // RB,),
        in_specs=[
            pl.BlockSpec((2, RB, D), lambda i: (0, i, 0)),
            pl.BlockSpec((1, D), lambda i: (0, 0)),
            pl.BlockSpec((D, 1), lambda i: (0, 0)),
        ],
        out_specs=[
            pl.BlockSpec((RB, D), lambda i: (i, 0)),
            pl.BlockSpec((RB, 1), lambda i: (i, 0)),
        ],
        out_shape=[
            jax.ShapeDtypeStruct((N, D), jnp.float32),
            jax.ShapeDtypeStruct((N, 1), jnp.float32),
        ],
    )(parts, bg3.reshape(1, D), p_unit)


def _rank(rmin, rmax, cmin, cmax, s_col, b_col, i_col, s_row, b_row, i_row):
    nchunks = CP // 128

    def body(rmin_ref, rmax_ref, cmin_ref, cmax_ref,
             sc_ref, bc_ref, ic_ref, sr_ref, br_ref, ir_ref, w_ref):
        pid = pl.program_id(0)
        blo = rmin_ref[pid]
        bhi = rmax_ref[pid]
        s_c = sc_ref[...]
        b_c = bc_ref[...]
        i_c = ic_ref[...]

        def cbody(cc, carry):
            rank, cnt = carry
            active = jnp.logical_not(
                (cmin_ref[cc] > bhi) | (cmax_ref[cc] < blo))

            def compute(carry):
                rank, cnt = carry
                s_r = sr_ref[:, pl.ds(cc * 128, 128)]
                b_r = br_ref[:, pl.ds(cc * 128, 128)]
                i_r = ir_ref[:, pl.ds(cc * 128, 128)]
                same = b_c == b_r
                beats = (s_r > s_c) | ((s_r == s_c) & (i_r < i_c))
                rank = rank + jnp.sum((same & beats).astype(jnp.float32),
                                      axis=1, keepdims=True)
                cnt = cnt + jnp.sum(same.astype(jnp.float32),
                                    axis=1, keepdims=True)
                return rank, cnt

            return lax.cond(active, compute, lambda c: c, (rank, cnt))

        zero = jnp.zeros((RB, 1), jnp.float32)
        rank, cnt = lax.fori_loop(0, nchunks, cbody, (zero, zero))
        k = jnp.floor((7.0 * cnt + 9.0) / 10.0)
        keep = rank < k
        w_ref[...] = jnp.where(keep, s_c / jnp.maximum(k, 1.0), 0.0)

    return pl.pallas_call(
        body,
        grid=(N // RB,),
        in_specs=[
            pl.BlockSpec(memory_space=pltpu.SMEM),
            pl.BlockSpec(memory_space=pltpu.SMEM),
            pl.BlockSpec(memory_space=pltpu.SMEM),
            pl.BlockSpec(memory_space=pltpu.SMEM),
            pl.BlockSpec((RB, 1), lambda i: (i, 0)),
            pl.BlockSpec((RB, 1), lambda i: (i, 0)),
            pl.BlockSpec((RB, 1), lambda i: (i, 0)),
            pl.BlockSpec((1, CP), lambda i: (0, 0)),
            pl.BlockSpec((1, CP), lambda i: (0, 0)),
            pl.BlockSpec((1, CP), lambda i: (0, 0)),
        ],
        out_specs=pl.BlockSpec((RB, 1), lambda i: (i, 0)),
        out_shape=jax.ShapeDtypeStruct((N, 1), jnp.float32),
    )(rmin, rmax, cmin, cmax, s_col, b_col, i_col, s_row, b_row, i_row)


def _pool_mlp(h, w_col, b_col, Wm0, bm0, Wm1, bm1, Wm2, bm2):
    def body(h_ref, w_ref, b_ref, wm0_ref, bm0_ref, wm1_ref, bm1_ref,
             wm2_ref, bm2_ref, o_ref, acc):
        pid = pl.program_id(0)

        @pl.when(pid == 0)
        def _():
            acc[...] = jnp.zeros_like(acc)

        gids = lax.broadcasted_iota(jnp.int32, (1, G), 1).astype(jnp.float32)
        onehot = (b_ref[...] == gids).astype(jnp.float32)
        wh = h_ref[...] * w_ref[...]
        acc[...] += lax.dot_general(
            onehot, wh, (((0,), (0,)), ((), ())),
            preferred_element_type=jnp.float32)

        @pl.when(pid == N // RB - 1)
        def _():
            hg = acc[...]
            z = jax.nn.relu(
                jnp.dot(hg, wm0_ref[...], preferred_element_type=jnp.float32)
                + bm0_ref[...])
            z = jax.nn.relu(
                jnp.dot(z, wm1_ref[...], preferred_element_type=jnp.float32)
                + bm1_ref[...])
            o_ref[...] = (
                jnp.dot(z, wm2_ref[...], preferred_element_type=jnp.float32)
                + bm2_ref[...])

    return pl.pallas_call(
        body,
        grid=(N // RB,),
        in_specs=[
            pl.BlockSpec((RB, D), lambda i: (i, 0)),
            pl.BlockSpec((RB, 1), lambda i: (i, 0)),
            pl.BlockSpec((RB, 1), lambda i: (i, 0)),
            pl.BlockSpec((D, D // 2), lambda i: (0, 0)),
            pl.BlockSpec((1, D // 2), lambda i: (0, 0)),
            pl.BlockSpec((D // 2, D // 4), lambda i: (0, 0)),
            pl.BlockSpec((1, D // 4), lambda i: (0, 0)),
            pl.BlockSpec((D // 4, NCLASS), lambda i: (0, 0)),
            pl.BlockSpec((1, NCLASS), lambda i: (0, 0)),
        ],
        out_specs=pl.BlockSpec((G, NCLASS), lambda i: (0, 0)),
        out_shape=jax.ShapeDtypeStruct((G, NCLASS), jnp.float32),
        scratch_shapes=[pltpu.VMEM((G, D), jnp.float32)],
    )(h, w_col, b_col, Wm0, bm0.reshape(1, -1), Wm1, bm1.reshape(1, -1),
      Wm2, bm2.reshape(1, -1))


def kernel(x, edge_index, batch, W_emb, b_emb, Wg0, bg0, Wg1, bg1, Wg2, bg2,
           Wg3, bg3, p_topk, Wm0, bm0, Wm1, bm1, Wm2, bm2):
    src = edge_index[0]
    dst = edge_index[1]

    hw = _mm_embed(x, W_emb, b_emb, Wg0)
    for Wg, bg in ((Wg1, bg1), (Wg2, bg2), (Wg3, bg3)):
        parts = _sc_scatter(hw, src, dst)
        hw = _mm_mid(parts, bg, Wg)
    parts = _sc_scatter(hw, src, dst)

    p_unit = (p_topk / jnp.linalg.norm(p_topk)).reshape(D, 1)
    h3, s_col = _finalize(parts, bg3, p_unit)

    bf = batch.astype(jnp.float32)
    b_col = bf.reshape(N, 1)
    i_col = jnp.arange(N, dtype=jnp.float32).reshape(N, 1)
    s_row = jnp.concatenate(
        [s_col.reshape(1, N), jnp.full((1, CP - N), -2.0, jnp.float32)], axis=1)
    b_row = jnp.concatenate(
        [bf.reshape(1, N), jnp.full((1, CP - N), -1.0, jnp.float32)], axis=1)
    i_row = jnp.arange(CP, dtype=jnp.float32).reshape(1, CP)
    bi = batch.astype(jnp.int32)
    rmin = bi.reshape(N // RB, RB).min(axis=1)
    rmax = bi.reshape(N // RB, RB).max(axis=1)
    bp = jnp.concatenate([bi, jnp.full((CP - N,), -1, jnp.int32)])
    cmin = bp.reshape(CP // 128, 128).min(axis=1)
    cmax = bp.reshape(CP // 128, 128).max(axis=1)

    w_col = _rank(rmin, rmax, cmin, cmax, s_col, b_col, i_col,
                  s_row, b_row, i_row)
    return _pool_mlp(h3, w_col, b_col, Wm0, bm0, Wm1, bm1, Wm2, bm2)


# speedup vs baseline: 3.7182x; 1.4085x over previous
import functools

import jax
import jax.numpy as jnp
from jax import lax
from jax.experimental import pallas as pl
from jax.experimental.pallas import tpu as pltpu
from jax.experimental.pallas import tpu_sc as plsc

N = 10000
E = 320000
D = 128
G = 64
NCLASS = 10

RB = 2000
CP = 10240

_NSUB = 32
_EDGES_PER_SUB = E // _NSUB
_CHUNK = 80
_NCHUNK = _EDGES_PER_SUB // _CHUNK
_ROWS_PER_SUB = 624
_ZROWS = 48
_OROWS = 208
_TAIL0 = 16 * _ROWS_PER_SUB
_TAILROWS = N - _TAIL0


def _sc_scatter(hw, src, dst):
    mesh = plsc.VectorSubcoreMesh(core_axis_name="c", subcore_axis_name="s")

    @functools.partial(
        pl.kernel,
        out_type=jax.ShapeDtypeStruct((2, N, D), jnp.float32),
        mesh=mesh,
        scratch_types=(
            [pltpu.VMEM((_CHUNK,), jnp.int32) for _ in range(16)]
            + [pltpu.VMEM((_CHUNK, D), jnp.float32) for _ in range(4)]
            + [pltpu.VMEM((_ZROWS, D), jnp.float32),
               pltpu.VMEM_SHARED((N, D), jnp.float32)]
            + [pltpu.SemaphoreType.DMA for _ in range(12)]
        ),
    )
    def k(hw_hbm, src_hbm, dst_hbm, out_hbm,
          sa0_, sa1_, sa2_, sa3_, da0_, da1_, da2_, da3_,
          sb0_, sb1_, sb2_, sb3_, db0_, db1_, db2_, db3_,
          r0, r1, r2, r3, zbuf, acc,
          sg0, sg1, sg2, sg3, sa0, sa1, sa2, sa3, sb0, sb1, sb2, sb3):
        sia = [sa0_, sa1_, sa2_, sa3_]
        dia = [da0_, da1_, da2_, da3_]
        sib = [sb0_, sb1_, sb2_, sb3_]
        dib = [db0_, db1_, db2_, db3_]
        rows = [r0, r1, r2, r3]
        sg = [sg0, sg1, sg2, sg3]
        sa = [sa0, sa1, sa2, sa3]
        sb = [sb0, sb1, sb2, sb3]
        c = lax.axis_index("c")
        s = lax.axis_index("s")
        wid = c * 16 + s

        zv = jnp.zeros((16,), jnp.float32)

        def zrow(r, carry):
            for l in range(D // 16):
                zbuf[r, pl.ds(l * 16, 16)] = zv
            return carry

        lax.fori_loop(0, _ZROWS, zrow, 0)
        row0 = s * _ROWS_PER_SUB
        for j in range(_ROWS_PER_SUB // _ZROWS):
            pltpu.sync_copy(zbuf, acc.at[pl.ds(row0 + j * _ZROWS, _ZROWS)])

        @pl.when(s == 15)
        def _():
            pltpu.sync_copy(zbuf.at[pl.ds(0, _TAILROWS)],
                            acc.at[pl.ds(_TAIL0, _TAILROWS)])

        plsc.subcore_barrier()

        ebase = wid * _EDGES_PER_SUB
        ngroups = _NCHUNK // 8

        def sslice(ch):
            return src_hbm.at[pl.ds(ebase + ch * _CHUNK, _CHUNK)]

        def dslice(ch):
            return dst_hbm.at[pl.ds(ebase + ch * _CHUNK, _CHUNK)]

        def load_idx(ch, sref, dref, sem):
            pltpu.async_copy(sslice(ch), sref, sem)
            pltpu.async_copy(dslice(ch), dref, sem)

        def wait_idx(ch, sref, dref, sem):
            pltpu.make_async_copy(sslice(ch), sref, sem).wait()
            pltpu.make_async_copy(dslice(ch), dref, sem).wait()

        for l in range(4):
            pltpu.sync_copy(sslice(l), sia[l])
            pltpu.sync_copy(dslice(l), dia[l])
            pltpu.async_copy(hw_hbm.at[sia[l]], rows[l], sg[l])
        for l in range(4):
            load_idx(4 + l, sib[l], dib[l], sb[l])

        def body(g, carry):
            base = g * 8
            for l in range(4):
                pltpu.make_async_copy(hw_hbm.at[sia[l]], rows[l], sg[l]).wait()
                pltpu.sync_copy(rows[l], acc.at[dia[l]], add=True)
                wait_idx(base + 4 + l, sib[l], dib[l], sb[l])
                pltpu.async_copy(hw_hbm.at[sib[l]], rows[l], sg[l])
                load_idx(base + 8 + l, sia[l], dia[l], sa[l])
            for l in range(4):
                pltpu.make_async_copy(hw_hbm.at[sib[l]], rows[l], sg[l]).wait()
                pltpu.sync_copy(rows[l], acc.at[dib[l]], add=True)

                @pl.when(g < ngroups - 1)
                def _():
                    wait_idx(base + 8 + l, sia[l], dia[l], sa[l])
                    pltpu.async_copy(hw_hbm.at[sia[l]], rows[l], sg[l])
                    load_idx(base + 12 + l, sib[l], dib[l], sb[l])

            return carry

        lax.fori_loop(0, ngroups, body, 0)

        t0 = ngroups * 8
        for l in range(4):
            wait_idx(t0 + l, sia[l], dia[l], sa[l])
            pltpu.async_copy(hw_hbm.at[sia[l]], rows[l], sg[l])
        pltpu.sync_copy(sslice(t0 + 4), sib[0])
        pltpu.sync_copy(dslice(t0 + 4), dib[0])
        for l in range(4):
            pltpu.make_async_copy(hw_hbm.at[sia[l]], rows[l], sg[l]).wait()
            pltpu.sync_copy(rows[l], acc.at[dia[l]], add=True)
        pltpu.async_copy(hw_hbm.at[sib[0]], rows[0], sg[0]).wait()
        pltpu.sync_copy(rows[0], acc.at[dib[0]], add=True)

        plsc.subcore_barrier()

        for j in range(_ROWS_PER_SUB // _OROWS):
            r = row0 + j * _OROWS
            pltpu.sync_copy(acc.at[pl.ds(r, _OROWS)],
                            out_hbm.at[c, pl.ds(r, _OROWS)])

        @pl.when(s == 15)
        def _():
            pltpu.sync_copy(acc.at[pl.ds(_TAIL0, _TAILROWS)],
                            out_hbm.at[c, pl.ds(_TAIL0, _TAILROWS)])

    return k(hw, src, dst)


def _mm_embed(x, W_emb, b_emb, Wg0):
    def body(x_ref, we_ref, be_ref, wg_ref, o_ref):
        h = jnp.dot(x_ref[...], we_ref[...], preferred_element_type=jnp.float32)
        h = h + be_ref[...]
        o_ref[...] = jnp.dot(h, wg_ref[...], preferred_element_type=jnp.float32)

    return pl.pallas_call(
        body,
        grid=(N // RB,),
        in_specs=[
            pl.BlockSpec((RB, D), lambda i: (i, 0)),
            pl.BlockSpec((D, D), lambda i: (0, 0)),
            pl.BlockSpec((1, D), lambda i: (0, 0)),
            pl.BlockSpec((D, D), lambda i: (0, 0)),
        ],
        out_specs=pl.BlockSpec((RB, D), lambda i: (i, 0)),
        out_shape=jax.ShapeDtypeStruct((N, D), jnp.float32),
    )(x, W_emb, b_emb.reshape(1, D), Wg0)


def _mm_mid(parts, bg, Wg):
    def body(p_ref, bg_ref, wg_ref, o_ref):
        h = jax.nn.relu(p_ref[0] + p_ref[1] + bg_ref[...])
        o_ref[...] = jnp.dot(h, wg_ref[...], preferred_element_type=jnp.float32)

    return pl.pallas_call(
        body,
        grid=(N // RB,),
        in_specs=[
            pl.BlockSpec((2, RB, D), lambda i: (0, i, 0)),
            pl.BlockSpec((1, D), lambda i: (0, 0)),
            pl.BlockSpec((D, D), lambda i: (0, 0)),
        ],
        out_specs=pl.BlockSpec((RB, D), lambda i: (i, 0)),
        out_shape=jax.ShapeDtypeStruct((N, D), jnp.float32),
    )(parts, bg.reshape(1, D), Wg)


def _finalize(parts, bg3, p_unit):
    def body(p_ref, bg_ref, pu_ref, h_ref, s_ref):
        h = jax.nn.relu(p_ref[0] + p_ref[1] + bg_ref[...])
        h_ref[...] = h
        s_ref[...] = jnp.tanh(
            jnp.dot(h, pu_ref[...], preferred_element_type=jnp.float32))

    return pl.pallas_call(
        body,
        grid=(N // RB,),
        in_specs=[
            pl.BlockSpec((2, RB, D), lambda i: (0, i, 0)),
            pl.BlockSpec((1, D), lambda i: (0, 0)),
            pl.BlockSpec((D, 1), lambda i: (0, 0)),
        ],
        out_specs=[
            pl.BlockSpec((RB, D), lambda i: (i, 0)),
            pl.BlockSpec((RB, 1), lambda i: (i, 0)),
        ],
        out_shape=[
            jax.ShapeDtypeStruct((N, D), jnp.float32),
            jax.ShapeDtypeStruct((N, 1), jnp.float32),
        ],
    )(parts, bg3.reshape(1, D), p_unit)


def _rank(cs, ce, s_col, b_col, i_col, s_row, b_row, i_row):
    def body(cs_ref, ce_ref, sc_ref, bc_ref, ic_ref, sr_ref, br_ref, ir_ref,
             w_ref):
        pid = pl.program_id(0)
        s_c = sc_ref[...]
        b_c = bc_ref[...]
        i_c = ic_ref[...]
        one = jnp.float32(1.0)
        zero2 = jnp.zeros((RB, 128), jnp.float32)

        def cbody(cc, carry):
            rank2, cnt2 = carry
            s_r = sr_ref[:, pl.ds(cc * 128, 128)]
            b_r = br_ref[:, pl.ds(cc * 128, 128)]
            i_r = ir_ref[:, pl.ds(cc * 128, 128)]
            same = b_c == b_r
            beats = same & ((s_r > s_c) | ((s_r == s_c) & (i_r < i_c)))
            rank2 = rank2 + jnp.where(beats, one, 0.0)
            cnt2 = cnt2 + jnp.where(same, one, 0.0)
            return rank2, cnt2

        rank2, cnt2 = lax.fori_loop(cs_ref[pid], ce_ref[pid], cbody,
                                    (zero2, zero2))
        rank = jnp.sum(rank2, axis=1, keepdims=True)
        cnt = jnp.sum(cnt2, axis=1, keepdims=True)
        k = jnp.floor((7.0 * cnt + 9.0) / 10.0)
        keep = rank < k
        w_ref[...] = jnp.where(keep, s_c / jnp.maximum(k, 1.0), 0.0)

    return pl.pallas_call(
        body,
        grid=(N // RB,),
        in_specs=[
            pl.BlockSpec(memory_space=pltpu.SMEM),
            pl.BlockSpec(memory_space=pltpu.SMEM),
            pl.BlockSpec((RB, 1), lambda i: (i, 0)),
            pl.BlockSpec((RB, 1), lambda i: (i, 0)),
            pl.BlockSpec((RB, 1), lambda i: (i, 0)),
            pl.BlockSpec((1, CP), lambda i: (0, 0)),
            pl.BlockSpec((1, CP), lambda i: (0, 0)),
            pl.BlockSpec((1, CP), lambda i: (0, 0)),
        ],
        out_specs=pl.BlockSpec((RB, 1), lambda i: (i, 0)),
        out_shape=jax.ShapeDtypeStruct((N, 1), jnp.float32),
    )(cs, ce, s_col, b_col, i_col, s_row, b_row, i_row)


def _pool_mlp(h, w_col, b_col, Wm0, bm0, Wm1, bm1, Wm2, bm2):
    def body(h_ref, w_ref, b_ref, wm0_ref, bm0_ref, wm1_ref, bm1_ref,
             wm2_ref, bm2_ref, o_ref, acc):
        pid = pl.program_id(0)

        @pl.when(pid == 0)
        def _():
            acc[...] = jnp.zeros_like(acc)

        gids = lax.broadcasted_iota(jnp.int32, (1, G), 1).astype(jnp.float32)
        onehot = (b_ref[...] == gids).astype(jnp.float32)
        wh = h_ref[...] * w_ref[...]
        acc[...] += lax.dot_general(
            onehot, wh, (((0,), (0,)), ((), ())),
            preferred_element_type=jnp.float32)

        @pl.when(pid == N // RB - 1)
        def _():
            hg = acc[...]
            z = jax.nn.relu(
                jnp.dot(hg, wm0_ref[...], preferred_element_type=jnp.float32)
                + bm0_ref[...])
            z = jax.nn.relu(
                jnp.dot(z, wm1_ref[...], preferred_element_type=jnp.float32)
                + bm1_ref[...])
            o_ref[...] = (
                jnp.dot(z, wm2_ref[...], preferred_element_type=jnp.float32)
                + bm2_ref[...])

    return pl.pallas_call(
        body,
        grid=(N // RB,),
        in_specs=[
            pl.BlockSpec((RB, D), lambda i: (i, 0)),
            pl.BlockSpec((RB, 1), lambda i: (i, 0)),
            pl.BlockSpec((RB, 1), lambda i: (i, 0)),
            pl.BlockSpec((D, D // 2), lambda i: (0, 0)),
            pl.BlockSpec((1, D // 2), lambda i: (0, 0)),
            pl.BlockSpec((D // 2, D // 4), lambda i: (0, 0)),
            pl.BlockSpec((1, D // 4), lambda i: (0, 0)),
            pl.BlockSpec((D // 4, NCLASS), lambda i: (0, 0)),
            pl.BlockSpec((1, NCLASS), lambda i: (0, 0)),
        ],
        out_specs=pl.BlockSpec((G, NCLASS), lambda i: (0, 0)),
        out_shape=jax.ShapeDtypeStruct((G, NCLASS), jnp.float32),
        scratch_shapes=[pltpu.VMEM((G, D), jnp.float32)],
    )(h, w_col, b_col, Wm0, bm0.reshape(1, -1), Wm1, bm1.reshape(1, -1),
      Wm2, bm2.reshape(1, -1))


def kernel(x, edge_index, batch, W_emb, b_emb, Wg0, bg0, Wg1, bg1, Wg2, bg2,
           Wg3, bg3, p_topk, Wm0, bm0, Wm1, bm1, Wm2, bm2):
    src = edge_index[0]
    dst = edge_index[1]

    hw = _mm_embed(x, W_emb, b_emb, Wg0)
    for Wg, bg in ((Wg1, bg1), (Wg2, bg2), (Wg3, bg3)):
        parts = _sc_scatter(hw, src, dst)
        hw = _mm_mid(parts, bg, Wg)
    parts = _sc_scatter(hw, src, dst)

    p_unit = (p_topk / jnp.linalg.norm(p_topk)).reshape(D, 1)
    h3, s_col = _finalize(parts, bg3, p_unit)

    bf = batch.astype(jnp.float32)
    b_col = bf.reshape(N, 1)
    i_col = jnp.arange(N, dtype=jnp.float32).reshape(N, 1)
    s_row = jnp.concatenate(
        [s_col.reshape(1, N), jnp.full((1, CP - N), -2.0, jnp.float32)], axis=1)
    b_row = jnp.concatenate(
        [bf.reshape(1, N), jnp.full((1, CP - N), -1.0, jnp.float32)], axis=1)
    i_row = jnp.arange(CP, dtype=jnp.float32).reshape(1, CP)
    bi = batch.astype(jnp.int32)
    starts = jnp.searchsorted(bi, jnp.arange(G + 1, dtype=jnp.int32))
    starts = starts.astype(jnp.int32)
    lo = bi.reshape(N // RB, RB)[:, 0]
    hi = bi.reshape(N // RB, RB)[:, -1]
    cs = starts[lo] // 128
    ce = (starts[hi + 1] + 127) // 128

    w_col = _rank(cs, ce, s_col, b_col, i_col, s_row, b_row, i_row)
    return _pool_mlp(h3, w_col, b_col, Wm0, bm0, Wm1, bm1, Wm2, bm2)


# speedup vs baseline: 3.9000x; 1.0489x over previous
import functools

import jax
import jax.numpy as jnp
from jax import lax
from jax.experimental import pallas as pl
from jax.experimental.pallas import tpu as pltpu
from jax.experimental.pallas import tpu_sc as plsc

N = 10000
E = 320000
D = 128
G = 64
NCLASS = 10

RB = 2000
CP = 10240

_NSUB = 32
_EDGES_PER_SUB = E // _NSUB
_CHUNK = 80
_NCHUNK = _EDGES_PER_SUB // _CHUNK
_ROWS_PER_SUB = 624
_ZROWS = 48
_OROWS = 208
_TAIL0 = 16 * _ROWS_PER_SUB
_TAILROWS = N - _TAIL0


def _sc_scatter(hw, src, dst):
    mesh = plsc.VectorSubcoreMesh(core_axis_name="c", subcore_axis_name="s")

    @functools.partial(
        pl.kernel,
        out_type=jax.ShapeDtypeStruct((2, N, D), jnp.float32),
        mesh=mesh,
        scratch_types=(
            [pltpu.VMEM((_CHUNK,), jnp.int32) for _ in range(16)]
            + [pltpu.VMEM((_CHUNK, D), jnp.float32) for _ in range(4)]
            + [pltpu.VMEM((_ZROWS, D), jnp.float32),
               pltpu.VMEM_SHARED((N, D), jnp.float32)]
            + [pltpu.SemaphoreType.DMA for _ in range(12)]
        ),
    )
    def k(hw_hbm, src_hbm, dst_hbm, out_hbm,
          sa0_, sa1_, sa2_, sa3_, da0_, da1_, da2_, da3_,
          sb0_, sb1_, sb2_, sb3_, db0_, db1_, db2_, db3_,
          r0, r1, r2, r3, zbuf, acc,
          sg0, sg1, sg2, sg3, sa0, sa1, sa2, sa3, sb0, sb1, sb2, sb3):
        sia = [sa0_, sa1_, sa2_, sa3_]
        dia = [da0_, da1_, da2_, da3_]
        sib = [sb0_, sb1_, sb2_, sb3_]
        dib = [db0_, db1_, db2_, db3_]
        rows = [r0, r1, r2, r3]
        sg = [sg0, sg1, sg2, sg3]
        sa = [sa0, sa1, sa2, sa3]
        sb = [sb0, sb1, sb2, sb3]
        c = lax.axis_index("c")
        s = lax.axis_index("s")
        wid = c * 16 + s

        zv = jnp.zeros((16,), jnp.float32)

        def zrow(r, carry):
            for l in range(D // 16):
                zbuf[r, pl.ds(l * 16, 16)] = zv
            return carry

        lax.fori_loop(0, _ZROWS, zrow, 0)
        row0 = s * _ROWS_PER_SUB
        for j in range(_ROWS_PER_SUB // _ZROWS):
            pltpu.sync_copy(zbuf, acc.at[pl.ds(row0 + j * _ZROWS, _ZROWS)])

        @pl.when(s == 15)
        def _():
            pltpu.sync_copy(zbuf.at[pl.ds(0, _TAILROWS)],
                            acc.at[pl.ds(_TAIL0, _TAILROWS)])

        plsc.subcore_barrier()

        ebase = wid * _EDGES_PER_SUB
        ngroups = _NCHUNK // 8

        def sslice(ch):
            return src_hbm.at[pl.ds(ebase + ch * _CHUNK, _CHUNK)]

        def dslice(ch):
            return dst_hbm.at[pl.ds(ebase + ch * _CHUNK, _CHUNK)]

        def load_idx(ch, sref, dref, sem):
            pltpu.async_copy(sslice(ch), sref, sem)
            pltpu.async_copy(dslice(ch), dref, sem)

        def wait_idx(ch, sref, dref, sem):
            pltpu.make_async_copy(sslice(ch), sref, sem).wait()
            pltpu.make_async_copy(dslice(ch), dref, sem).wait()

        for l in range(4):
            pltpu.sync_copy(sslice(l), sia[l])
            pltpu.sync_copy(dslice(l), dia[l])
            pltpu.async_copy(hw_hbm.at[sia[l]], rows[l], sg[l])
        for l in range(4):
            load_idx(4 + l, sib[l], dib[l], sb[l])

        def body(g, carry):
            base = g * 8
            for l in range(4):
                pltpu.make_async_copy(hw_hbm.at[sia[l]], rows[l], sg[l]).wait()
                pltpu.sync_copy(rows[l], acc.at[dia[l]], add=True)
                wait_idx(base + 4 + l, sib[l], dib[l], sb[l])
                pltpu.async_copy(hw_hbm.at[sib[l]], rows[l], sg[l])
                load_idx(base + 8 + l, sia[l], dia[l], sa[l])
            for l in range(4):
                pltpu.make_async_copy(hw_hbm.at[sib[l]], rows[l], sg[l]).wait()
                pltpu.sync_copy(rows[l], acc.at[dib[l]], add=True)

                @pl.when(g < ngroups - 1)
                def _():
                    wait_idx(base + 8 + l, sia[l], dia[l], sa[l])
                    pltpu.async_copy(hw_hbm.at[sia[l]], rows[l], sg[l])
                    load_idx(base + 12 + l, sib[l], dib[l], sb[l])

            return carry

        lax.fori_loop(0, ngroups, body, 0)

        t0 = ngroups * 8
        for l in range(4):
            wait_idx(t0 + l, sia[l], dia[l], sa[l])
            pltpu.async_copy(hw_hbm.at[sia[l]], rows[l], sg[l])
        pltpu.sync_copy(sslice(t0 + 4), sib[0])
        pltpu.sync_copy(dslice(t0 + 4), dib[0])
        for l in range(4):
            pltpu.make_async_copy(hw_hbm.at[sia[l]], rows[l], sg[l]).wait()
            pltpu.sync_copy(rows[l], acc.at[dia[l]], add=True)
        pltpu.async_copy(hw_hbm.at[sib[0]], rows[0], sg[0]).wait()
        pltpu.sync_copy(rows[0], acc.at[dib[0]], add=True)

        plsc.subcore_barrier()

        for j in range(_ROWS_PER_SUB // _OROWS):
            r = row0 + j * _OROWS
            pltpu.sync_copy(acc.at[pl.ds(r, _OROWS)],
                            out_hbm.at[c, pl.ds(r, _OROWS)])

        @pl.when(s == 15)
        def _():
            pltpu.sync_copy(acc.at[pl.ds(_TAIL0, _TAILROWS)],
                            out_hbm.at[c, pl.ds(_TAIL0, _TAILROWS)])

    return k(hw, src, dst)


def _mm_embed(x, W_emb, b_emb, Wg0):
    def body(x_ref, we_ref, be_ref, wg_ref, o_ref):
        h = jnp.dot(x_ref[...], we_ref[...], preferred_element_type=jnp.float32)
        h = h + be_ref[...]
        o_ref[...] = jnp.dot(h, wg_ref[...], preferred_element_type=jnp.float32)

    return pl.pallas_call(
        body,
        grid=(N // RB,),
        in_specs=[
            pl.BlockSpec((RB, D), lambda i: (i, 0)),
            pl.BlockSpec((D, D), lambda i: (0, 0)),
            pl.BlockSpec((1, D), lambda i: (0, 0)),
            pl.BlockSpec((D, D), lambda i: (0, 0)),
        ],
        out_specs=pl.BlockSpec((RB, D), lambda i: (i, 0)),
        out_shape=jax.ShapeDtypeStruct((N, D), jnp.float32),
    )(x, W_emb, b_emb.reshape(1, D), Wg0)


def _mm_mid(parts, bg, Wg):
    def body(p_ref, bg_ref, wg_ref, o_ref):
        h = jax.nn.relu(p_ref[0] + p_ref[1] + bg_ref[...])
        o_ref[...] = jnp.dot(h, wg_ref[...], preferred_element_type=jnp.float32)

    return pl.pallas_call(
        body,
        grid=(N // RB,),
        in_specs=[
            pl.BlockSpec((2, RB, D), lambda i: (0, i, 0)),
            pl.BlockSpec((1, D), lambda i: (0, 0)),
            pl.BlockSpec((D, D), lambda i: (0, 0)),
        ],
        out_specs=pl.BlockSpec((RB, D), lambda i: (i, 0)),
        out_shape=jax.ShapeDtypeStruct((N, D), jnp.float32),
    )(parts, bg.reshape(1, D), Wg)


def _finalize(parts, bg3, p_topk):
    def body(p_ref, bg_ref, pu_ref, h_ref, s_ref):
        h = jax.nn.relu(p_ref[0] + p_ref[1] + bg_ref[...])
        h_ref[...] = h
        p = pu_ref[...]
        p_unit = p * lax.rsqrt(jnp.sum(p * p))
        s_ref[...] = jnp.tanh(
            jnp.dot(h, p_unit, preferred_element_type=jnp.float32))

    return pl.pallas_call(
        body,
        grid=(N // RB,),
        in_specs=[
            pl.BlockSpec((2, RB, D), lambda i: (0, i, 0)),
            pl.BlockSpec((1, D), lambda i: (0, 0)),
            pl.BlockSpec((D, 1), lambda i: (0, 0)),
        ],
        out_specs=[
            pl.BlockSpec((RB, D), lambda i: (i, 0)),
            pl.BlockSpec((RB, 1), lambda i: (i, 0)),
        ],
        out_shape=[
            jax.ShapeDtypeStruct((N, D), jnp.float32),
            jax.ShapeDtypeStruct((N, 1), jnp.float32),
        ],
    )(parts, bg3.reshape(1, D), p_topk)


def _rank(cs, ce, s_col, b_col, s_row, b_row):
    def body(cs_ref, ce_ref, sc_ref, bc_ref, sr_ref, br_ref, w_ref):
        pid = pl.program_id(0)
        s_c = sc_ref[...]
        b_c = bc_ref[...]
        i_c = (jnp.float32(RB) * pid.astype(jnp.float32)
               + lax.broadcasted_iota(jnp.int32, (RB, 1), 0)
               .astype(jnp.float32))
        lane = lax.broadcasted_iota(jnp.int32, (1, 128), 1).astype(jnp.float32)
        one = jnp.float32(1.0)
        zero2 = jnp.zeros((RB, 128), jnp.float32)

        def cbody(cc, carry):
            rank2, cnt2 = carry
            s_r = sr_ref[:, pl.ds(cc * 128, 128)]
            b_r = br_ref[:, pl.ds(cc * 128, 128)]
            i_r = jnp.float32(128.0) * cc.astype(jnp.float32) + lane
            same = b_c == b_r
            beats = same & ((s_r > s_c) | ((s_r == s_c) & (i_r < i_c)))
            rank2 = rank2 + jnp.where(beats, one, 0.0)
            cnt2 = cnt2 + jnp.where(same, one, 0.0)
            return rank2, cnt2

        rank2, cnt2 = lax.fori_loop(cs_ref[pid], ce_ref[pid], cbody,
                                    (zero2, zero2))
        rank = jnp.sum(rank2, axis=1, keepdims=True)
        cnt = jnp.sum(cnt2, axis=1, keepdims=True)
        k = jnp.floor((7.0 * cnt + 9.0) / 10.0)
        keep = rank < k
        w_ref[...] = jnp.where(keep, s_c / jnp.maximum(k, 1.0), 0.0)

    return pl.pallas_call(
        body,
        grid=(N // RB,),
        in_specs=[
            pl.BlockSpec(memory_space=pltpu.SMEM),
            pl.BlockSpec(memory_space=pltpu.SMEM),
            pl.BlockSpec((RB, 1), lambda i: (i, 0)),
            pl.BlockSpec((RB, 1), lambda i: (i, 0)),
            pl.BlockSpec((1, CP), lambda i: (0, 0)),
            pl.BlockSpec((1, CP), lambda i: (0, 0)),
        ],
        out_specs=pl.BlockSpec((RB, 1), lambda i: (i, 0)),
        out_shape=jax.ShapeDtypeStruct((N, 1), jnp.float32),
    )(cs, ce, s_col, b_col, s_row, b_row)


def _pool_mlp(h, w_col, b_col, Wm0, bm0, Wm1, bm1, Wm2, bm2):
    def body(h_ref, w_ref, b_ref, wm0_ref, bm0_ref, wm1_ref, bm1_ref,
             wm2_ref, bm2_ref, o_ref, acc):
        pid = pl.program_id(0)

        @pl.when(pid == 0)
        def _():
            acc[...] = jnp.zeros_like(acc)

        gids = lax.broadcasted_iota(jnp.int32, (1, G), 1).astype(jnp.float32)
        onehot = (b_ref[...] == gids).astype(jnp.float32)
        wh = h_ref[...] * w_ref[...]
        acc[...] += lax.dot_general(
            onehot, wh, (((0,), (0,)), ((), ())),
            preferred_element_type=jnp.float32)

        @pl.when(pid == N // RB - 1)
        def _():
            hg = acc[...]
            z = jax.nn.relu(
                jnp.dot(hg, wm0_ref[...], preferred_element_type=jnp.float32)
                + bm0_ref[...])
            z = jax.nn.relu(
                jnp.dot(z, wm1_ref[...], preferred_element_type=jnp.float32)
                + bm1_ref[...])
            o_ref[...] = (
                jnp.dot(z, wm2_ref[...], preferred_element_type=jnp.float32)
                + bm2_ref[...])

    return pl.pallas_call(
        body,
        grid=(N // RB,),
        in_specs=[
            pl.BlockSpec((RB, D), lambda i: (i, 0)),
            pl.BlockSpec((RB, 1), lambda i: (i, 0)),
            pl.BlockSpec((RB, 1), lambda i: (i, 0)),
            pl.BlockSpec((D, D // 2), lambda i: (0, 0)),
            pl.BlockSpec((1, D // 2), lambda i: (0, 0)),
            pl.BlockSpec((D // 2, D // 4), lambda i: (0, 0)),
            pl.BlockSpec((1, D // 4), lambda i: (0, 0)),
            pl.BlockSpec((D // 4, NCLASS), lambda i: (0, 0)),
            pl.BlockSpec((1, NCLASS), lambda i: (0, 0)),
        ],
        out_specs=pl.BlockSpec((G, NCLASS), lambda i: (0, 0)),
        out_shape=jax.ShapeDtypeStruct((G, NCLASS), jnp.float32),
        scratch_shapes=[pltpu.VMEM((G, D), jnp.float32)],
    )(h, w_col, b_col, Wm0, bm0.reshape(1, -1), Wm1, bm1.reshape(1, -1),
      Wm2, bm2.reshape(1, -1))


def kernel(x, edge_index, batch, W_emb, b_emb, Wg0, bg0, Wg1, bg1, Wg2, bg2,
           Wg3, bg3, p_topk, Wm0, bm0, Wm1, bm1, Wm2, bm2):
    src = edge_index[0]
    dst = edge_index[1]

    hw = _mm_embed(x, W_emb, b_emb, Wg0)
    for Wg, bg in ((Wg1, bg1), (Wg2, bg2), (Wg3, bg3)):
        parts = _sc_scatter(hw, src, dst)
        hw = _mm_mid(parts, bg, Wg)
    parts = _sc_scatter(hw, src, dst)

    h3, s_col = _finalize(parts, bg3, p_topk.reshape(D, 1))

    bf = batch.astype(jnp.float32)
    b_col = bf.reshape(N, 1)
    s_row = jnp.concatenate(
        [s_col.reshape(1, N), jnp.full((1, CP - N), -2.0, jnp.float32)], axis=1)
    b_row = jnp.concatenate(
        [bf.reshape(1, N), jnp.full((1, CP - N), -1.0, jnp.float32)], axis=1)
    bi = batch.astype(jnp.int32)
    starts = jnp.searchsorted(bi, jnp.arange(G + 1, dtype=jnp.int32))
    starts = starts.astype(jnp.int32)
    lo = bi.reshape(N // RB, RB)[:, 0]
    hi = bi.reshape(N // RB, RB)[:, -1]
    cs = starts[lo] // 128
    ce = (starts[hi + 1] + 127) // 128

    w_col = _rank(cs, ce, s_col, b_col, s_row, b_row)
    return _pool_mlp(h3, w_col, b_col, Wm0, bm0, Wm1, bm1, Wm2, bm2)


# speedup vs baseline: 4.1652x; 1.0680x over previous
import functools

import jax
import jax.numpy as jnp
from jax import lax
from jax.experimental import pallas as pl
from jax.experimental.pallas import tpu as pltpu
from jax.experimental.pallas import tpu_sc as plsc

N = 10000
E = 320000
D = 128
G = 64
NCLASS = 10

RB = 2000
RBR = 1000
CP = 10240

_NSUB = 32
_EDGES_PER_SUB = E // _NSUB
_CHUNK = 80
_NCHUNK = _EDGES_PER_SUB // _CHUNK
_ROWS_PER_SUB = 624
_ZROWS = 48
_OROWS = 208
_TAIL0 = 16 * _ROWS_PER_SUB
_TAILROWS = N - _TAIL0


def _sc_scatter(hw, src, dst):
    mesh = plsc.VectorSubcoreMesh(core_axis_name="c", subcore_axis_name="s")

    @functools.partial(
        pl.kernel,
        out_type=jax.ShapeDtypeStruct((2, N, D), jnp.float32),
        mesh=mesh,
        scratch_types=(
            [pltpu.VMEM((_CHUNK,), jnp.int32) for _ in range(16)]
            + [pltpu.VMEM((_CHUNK, D), jnp.float32) for _ in range(4)]
            + [pltpu.VMEM((_ZROWS, D), jnp.float32),
               pltpu.VMEM_SHARED((N, D), jnp.float32)]
            + [pltpu.SemaphoreType.DMA for _ in range(12)]
        ),
    )
    def k(hw_hbm, src_hbm, dst_hbm, out_hbm,
          sa0_, sa1_, sa2_, sa3_, da0_, da1_, da2_, da3_,
          sb0_, sb1_, sb2_, sb3_, db0_, db1_, db2_, db3_,
          r0, r1, r2, r3, zbuf, acc,
          sg0, sg1, sg2, sg3, sa0, sa1, sa2, sa3, sb0, sb1, sb2, sb3):
        sia = [sa0_, sa1_, sa2_, sa3_]
        dia = [da0_, da1_, da2_, da3_]
        sib = [sb0_, sb1_, sb2_, sb3_]
        dib = [db0_, db1_, db2_, db3_]
        rows = [r0, r1, r2, r3]
        sg = [sg0, sg1, sg2, sg3]
        sa = [sa0, sa1, sa2, sa3]
        sb = [sb0, sb1, sb2, sb3]
        c = lax.axis_index("c")
        s = lax.axis_index("s")
        wid = c * 16 + s

        zv = jnp.zeros((16,), jnp.float32)

        def zrow(r, carry):
            for l in range(D // 16):
                zbuf[r, pl.ds(l * 16, 16)] = zv
            return carry

        lax.fori_loop(0, _ZROWS, zrow, 0)
        row0 = s * _ROWS_PER_SUB
        for j in range(_ROWS_PER_SUB // _ZROWS):
            pltpu.sync_copy(zbuf, acc.at[pl.ds(row0 + j * _ZROWS, _ZROWS)])

        @pl.when(s == 15)
        def _():
            pltpu.sync_copy(zbuf.at[pl.ds(0, _TAILROWS)],
                            acc.at[pl.ds(_TAIL0, _TAILROWS)])

        plsc.subcore_barrier()

        ebase = wid * _EDGES_PER_SUB
        ngroups = _NCHUNK // 8

        def sslice(ch):
            return src_hbm.at[pl.ds(ebase + ch * _CHUNK, _CHUNK)]

        def dslice(ch):
            return dst_hbm.at[pl.ds(ebase + ch * _CHUNK, _CHUNK)]

        def load_idx(ch, sref, dref, sem):
            pltpu.async_copy(sslice(ch), sref, sem)
            pltpu.async_copy(dslice(ch), dref, sem)

        def wait_idx(ch, sref, dref, sem):
            pltpu.make_async_copy(sslice(ch), sref, sem).wait()
            pltpu.make_async_copy(dslice(ch), dref, sem).wait()

        for l in range(4):
            pltpu.sync_copy(sslice(l), sia[l])
            pltpu.sync_copy(dslice(l), dia[l])
            pltpu.async_copy(hw_hbm.at[sia[l]], rows[l], sg[l])
        for l in range(4):
            load_idx(4 + l, sib[l], dib[l], sb[l])

        def body(g, carry):
            base = g * 8
            for l in range(4):
                pltpu.make_async_copy(hw_hbm.at[sia[l]], rows[l], sg[l]).wait()
                pltpu.sync_copy(rows[l], acc.at[dia[l]], add=True)
                wait_idx(base + 4 + l, sib[l], dib[l], sb[l])
                pltpu.async_copy(hw_hbm.at[sib[l]], rows[l], sg[l])
                load_idx(base + 8 + l, sia[l], dia[l], sa[l])
            for l in range(4):
                pltpu.make_async_copy(hw_hbm.at[sib[l]], rows[l], sg[l]).wait()
                pltpu.sync_copy(rows[l], acc.at[dib[l]], add=True)

                @pl.when(g < ngroups - 1)
                def _():
                    wait_idx(base + 8 + l, sia[l], dia[l], sa[l])
                    pltpu.async_copy(hw_hbm.at[sia[l]], rows[l], sg[l])
                    load_idx(base + 12 + l, sib[l], dib[l], sb[l])

            return carry

        lax.fori_loop(0, ngroups, body, 0)

        t0 = ngroups * 8
        for l in range(4):
            wait_idx(t0 + l, sia[l], dia[l], sa[l])
            pltpu.async_copy(hw_hbm.at[sia[l]], rows[l], sg[l])
        pltpu.sync_copy(sslice(t0 + 4), sib[0])
        pltpu.sync_copy(dslice(t0 + 4), dib[0])
        for l in range(4):
            pltpu.make_async_copy(hw_hbm.at[sia[l]], rows[l], sg[l]).wait()
            pltpu.sync_copy(rows[l], acc.at[dia[l]], add=True)
        pltpu.async_copy(hw_hbm.at[sib[0]], rows[0], sg[0]).wait()
        pltpu.sync_copy(rows[0], acc.at[dib[0]], add=True)

        plsc.subcore_barrier()

        for j in range(_ROWS_PER_SUB // _OROWS):
            r = row0 + j * _OROWS
            pltpu.sync_copy(acc.at[pl.ds(r, _OROWS)],
                            out_hbm.at[c, pl.ds(r, _OROWS)])

        @pl.when(s == 15)
        def _():
            pltpu.sync_copy(acc.at[pl.ds(_TAIL0, _TAILROWS)],
                            out_hbm.at[c, pl.ds(_TAIL0, _TAILROWS)])

    return k(hw, src, dst)


def _mm_embed(x, W_emb, b_emb, Wg0):
    def body(x_ref, we_ref, be_ref, wg_ref, o_ref):
        h = jnp.dot(x_ref[...], we_ref[...], preferred_element_type=jnp.float32)
        h = h + be_ref[...]
        o_ref[...] = jnp.dot(h, wg_ref[...], preferred_element_type=jnp.float32)

    return pl.pallas_call(
        body,
        grid=(N // RB,),
        in_specs=[
            pl.BlockSpec((RB, D), lambda i: (i, 0)),
            pl.BlockSpec((D, D), lambda i: (0, 0)),
            pl.BlockSpec((1, D), lambda i: (0, 0)),
            pl.BlockSpec((D, D), lambda i: (0, 0)),
        ],
        out_specs=pl.BlockSpec((RB, D), lambda i: (i, 0)),
        out_shape=jax.ShapeDtypeStruct((N, D), jnp.float32),
    )(x, W_emb, b_emb.reshape(1, D), Wg0)


def _mm_mid(parts, bg, Wg):
    def body(p_ref, bg_ref, wg_ref, o_ref):
        h = jax.nn.relu(p_ref[0] + p_ref[1] + bg_ref[...])
        o_ref[...] = jnp.dot(h, wg_ref[...], preferred_element_type=jnp.float32)

    return pl.pallas_call(
        body,
        grid=(N // RB,),
        in_specs=[
            pl.BlockSpec((2, RB, D), lambda i: (0, i, 0)),
            pl.BlockSpec((1, D), lambda i: (0, 0)),
            pl.BlockSpec((D, D), lambda i: (0, 0)),
        ],
        out_specs=pl.BlockSpec((RB, D), lambda i: (i, 0)),
        out_shape=jax.ShapeDtypeStruct((N, D), jnp.float32),
    )(parts, bg.reshape(1, D), Wg)


def _finalize(parts, bg3, p_topk):
    def body(p_ref, bg_ref, pu_ref, h_ref, s_ref):
        h = jax.nn.relu(p_ref[0] + p_ref[1] + bg_ref[...])
        h_ref[...] = h
        p = pu_ref[...]
        p_unit = p * lax.rsqrt(jnp.sum(p * p))
        s_ref[...] = jnp.tanh(
            jnp.dot(h, p_unit, preferred_element_type=jnp.float32))

    return pl.pallas_call(
        body,
        grid=(N // RB,),
        in_specs=[
            pl.BlockSpec((2, RB, D), lambda i: (0, i, 0)),
            pl.BlockSpec((1, D), lambda i: (0, 0)),
            pl.BlockSpec((D, 1), lambda i: (0, 0)),
        ],
        out_specs=[
            pl.BlockSpec((RB, D), lambda i: (i, 0)),
            pl.BlockSpec((RB, 1), lambda i: (i, 0)),
        ],
        out_shape=[
            jax.ShapeDtypeStruct((N, D), jnp.float32),
            jax.ShapeDtypeStruct((N, 1), jnp.float32),
        ],
    )(parts, bg3.reshape(1, D), p_topk)


def _rank(cs, ce, s_col, b_col, s_row, b_row):
    def body(cs_ref, ce_ref, sc_ref, bc_ref, sr_ref, br_ref, w_ref):
        pid = pl.program_id(0)
        s_c = sc_ref[...]
        b_c = bc_ref[...]
        i_c = (jnp.float32(RBR) * pid.astype(jnp.float32)
               + lax.broadcasted_iota(jnp.int32, (RBR, 1), 0)
               .astype(jnp.float32))
        lane = lax.broadcasted_iota(jnp.int32, (1, 128), 1).astype(jnp.float32)
        one = jnp.float32(1.0)
        zero2 = jnp.zeros((RBR, 128), jnp.float32)

        def cbody(cc, carry):
            rank2, cnt2 = carry
            s_r = sr_ref[:, pl.ds(cc * 128, 128)]
            b_r = br_ref[:, pl.ds(cc * 128, 128)]
            i_r = jnp.float32(128.0) * cc.astype(jnp.float32) + lane
            same = b_c == b_r
            beats = same & ((s_r > s_c) | ((s_r == s_c) & (i_r < i_c)))
            rank2 = rank2 + jnp.where(beats, one, 0.0)
            cnt2 = cnt2 + jnp.where(same, one, 0.0)
            return rank2, cnt2

        rank2, cnt2 = lax.fori_loop(cs_ref[pid], ce_ref[pid], cbody,
                                    (zero2, zero2))
        rank = jnp.sum(rank2, axis=1, keepdims=True)
        cnt = jnp.sum(cnt2, axis=1, keepdims=True)
        k = jnp.floor((7.0 * cnt + 9.0) / 10.0)
        keep = rank < k
        w_ref[...] = jnp.where(keep, s_c / jnp.maximum(k, 1.0), 0.0)

    return pl.pallas_call(
        body,
        grid=(N // RBR,),
        in_specs=[
            pl.BlockSpec(memory_space=pltpu.SMEM),
            pl.BlockSpec(memory_space=pltpu.SMEM),
            pl.BlockSpec((RBR, 1), lambda i: (i, 0)),
            pl.BlockSpec((RBR, 1), lambda i: (i, 0)),
            pl.BlockSpec((1, CP), lambda i: (0, 0)),
            pl.BlockSpec((1, CP), lambda i: (0, 0)),
        ],
        out_specs=pl.BlockSpec((RBR, 1), lambda i: (i, 0)),
        out_shape=jax.ShapeDtypeStruct((N, 1), jnp.float32),
    )(cs, ce, s_col, b_col, s_row, b_row)


def _pool_mlp(h, w_col, b_col, Wm0, bm0, Wm1, bm1, Wm2, bm2):
    def body(h_ref, w_ref, b_ref, wm0_ref, bm0_ref, wm1_ref, bm1_ref,
             wm2_ref, bm2_ref, o_ref, acc):
        pid = pl.program_id(0)

        @pl.when(pid == 0)
        def _():
            acc[...] = jnp.zeros_like(acc)

        gids = lax.broadcasted_iota(jnp.int32, (1, G), 1).astype(jnp.float32)
        onehot = (b_ref[...] == gids).astype(jnp.float32)
        wh = h_ref[...] * w_ref[...]
        acc[...] += lax.dot_general(
            onehot, wh, (((0,), (0,)), ((), ())),
            preferred_element_type=jnp.float32)

        @pl.when(pid == N // RB - 1)
        def _():
            hg = acc[...]
            z = jax.nn.relu(
                jnp.dot(hg, wm0_ref[...], preferred_element_type=jnp.float32)
                + bm0_ref[...])
            z = jax.nn.relu(
                jnp.dot(z, wm1_ref[...], preferred_element_type=jnp.float32)
                + bm1_ref[...])
            o_ref[...] = (
                jnp.dot(z, wm2_ref[...], preferred_element_type=jnp.float32)
                + bm2_ref[...])

    return pl.pallas_call(
        body,
        grid=(N // RB,),
        in_specs=[
            pl.BlockSpec((RB, D), lambda i: (i, 0)),
            pl.BlockSpec((RB, 1), lambda i: (i, 0)),
            pl.BlockSpec((RB, 1), lambda i: (i, 0)),
            pl.BlockSpec((D, D // 2), lambda i: (0, 0)),
            pl.BlockSpec((1, D // 2), lambda i: (0, 0)),
            pl.BlockSpec((D // 2, D // 4), lambda i: (0, 0)),
            pl.BlockSpec((1, D // 4), lambda i: (0, 0)),
            pl.BlockSpec((D // 4, NCLASS), lambda i: (0, 0)),
            pl.BlockSpec((1, NCLASS), lambda i: (0, 0)),
        ],
        out_specs=pl.BlockSpec((G, NCLASS), lambda i: (0, 0)),
        out_shape=jax.ShapeDtypeStruct((G, NCLASS), jnp.float32),
        scratch_shapes=[pltpu.VMEM((G, D), jnp.float32)],
    )(h, w_col, b_col, Wm0, bm0.reshape(1, -1), Wm1, bm1.reshape(1, -1),
      Wm2, bm2.reshape(1, -1))


def kernel(x, edge_index, batch, W_emb, b_emb, Wg0, bg0, Wg1, bg1, Wg2, bg2,
           Wg3, bg3, p_topk, Wm0, bm0, Wm1, bm1, Wm2, bm2):
    src = edge_index[0]
    dst = edge_index[1]

    hw = _mm_embed(x, W_emb, b_emb, Wg0)
    for Wg, bg in ((Wg1, bg1), (Wg2, bg2), (Wg3, bg3)):
        parts = _sc_scatter(hw, src, dst)
        hw = _mm_mid(parts, bg, Wg)
    parts = _sc_scatter(hw, src, dst)

    h3, s_col = _finalize(parts, bg3, p_topk.reshape(D, 1))

    bf = batch.astype(jnp.float32)
    b_col = bf.reshape(N, 1)
    s_row = jnp.concatenate(
        [s_col.reshape(1, N), jnp.full((1, CP - N), -2.0, jnp.float32)], axis=1)
    b_row = jnp.concatenate(
        [bf.reshape(1, N), jnp.full((1, CP - N), -1.0, jnp.float32)], axis=1)
    bi = batch.astype(jnp.int32)
    starts = jnp.searchsorted(bi, jnp.arange(G + 1, dtype=jnp.int32))
    starts = starts.astype(jnp.int32)
    lo = bi.reshape(N // RBR, RBR)[:, 0]
    hi = bi.reshape(N // RBR, RBR)[:, -1]
    cs = starts[lo] // 128
    ce = (starts[hi + 1] + 127) // 128

    w_col = _rank(cs, ce, s_col, b_col, s_row, b_row)
    return _pool_mlp(h3, w_col, b_col, Wm0, bm0, Wm1, bm1, Wm2, bm2)
